# Initial kernel scaffold; baseline (speedup 1.0000x reference)
import functools

import jax
import jax.numpy as jnp
from jax import lax
from jax.experimental import pallas as pl
from jax.experimental.pallas import tpu as pltpu
from jax.experimental.pallas import tpu_sc as plsc

N = 10000
E = 320000
D_IN = 128
D_H = 128
D_OUT = 64

NC = 2
NS = 16
L = 16
NW = NC * NS

CHUNK = 128
NCHUNKS = E // CHUNK
NPAD = 10240
ROWS_PER_TILE = NPAD // NS

_vector_mesh = plsc.VectorSubcoreMesh(core_axis_name="c", subcore_axis_name="s")


@functools.partial(
    pl.kernel,
    out_type=(
        jax.ShapeDtypeStruct((NC, NPAD), jnp.float32),
        jax.ShapeDtypeStruct((NC, NPAD), jnp.float32),
    ),
    mesh=_vector_mesh,
    scratch_types=[
        pltpu.VMEM((NPAD,), jnp.float32),
        pltpu.VMEM((NPAD,), jnp.float32),
        pltpu.VMEM((CHUNK,), jnp.int32),
        pltpu.VMEM((ROWS_PER_TILE,), jnp.float32),
        pltpu.VMEM_SHARED((NS, NPAD), jnp.float32),
        pltpu.VMEM_SHARED((NS, NPAD), jnp.float32),
        pltpu.SemaphoreType.DMA,
    ],
)
def _sc_degrees(src_hbm, dst_hbm, dego_hbm, degi_hbm,
                ho, hi, idxb, tmp, sho, shi, sem):
    cid = lax.axis_index("c")
    sid = lax.axis_index("s")
    wid = cid * NS + sid

    zeros16 = jnp.zeros((L,), jnp.float32)
    ones16 = jnp.ones((L,), jnp.float32)

    @pl.loop(0, NPAD // L)
    def _(i):
        ho[pl.ds(i * L, L)] = zeros16
        hi[pl.ds(i * L, L)] = zeros16

    nfull = NCHUNKS // NW + (1 if NCHUNKS % NW else 0)

    @pl.loop(0, nfull)
    def _(c):
        g = c * NW + wid

        @pl.when(g < NCHUNKS)
        def _():
            pltpu.sync_copy(src_hbm.at[g], idxb)

            @pl.loop(0, CHUNK // L)
            def _(i):
                plsc.addupdate_scatter(ho, [idxb[pl.ds(i * L, L)]], ones16)

            pltpu.sync_copy(dst_hbm.at[g], idxb)

            @pl.loop(0, CHUNK // L)
            def _(i):
                plsc.addupdate_scatter(hi, [idxb[pl.ds(i * L, L)]], ones16)

    pltpu.sync_copy(ho, sho.at[sid])
    pltpu.sync_copy(hi, shi.at[sid])
    plsc.subcore_barrier()

    base = sid * ROWS_PER_TILE
    pltpu.sync_copy(sho.at[0, pl.ds(base, ROWS_PER_TILE)],
                    ho.at[pl.ds(0, ROWS_PER_TILE)])
    pltpu.sync_copy(shi.at[0, pl.ds(base, ROWS_PER_TILE)],
                    hi.at[pl.ds(0, ROWS_PER_TILE)])
    for t in range(1, NS):
        pltpu.sync_copy(sho.at[t, pl.ds(base, ROWS_PER_TILE)], tmp)

        @pl.loop(0, ROWS_PER_TILE // L)
        def _(j):
            ho[pl.ds(j * L, L)] = ho[pl.ds(j * L, L)] + tmp[pl.ds(j * L, L)]

        pltpu.sync_copy(shi.at[t, pl.ds(base, ROWS_PER_TILE)], tmp)

        @pl.loop(0, ROWS_PER_TILE // L)
        def _(j):
            hi[pl.ds(j * L, L)] = hi[pl.ds(j * L, L)] + tmp[pl.ds(j * L, L)]

    pltpu.sync_copy(ho.at[pl.ds(0, ROWS_PER_TILE)],
                    dego_hbm.at[cid, pl.ds(base, ROWS_PER_TILE)])
    pltpu.sync_copy(hi.at[pl.ds(0, ROWS_PER_TILE)],
                    degi_hbm.at[cid, pl.ds(base, ROWS_PER_TILE)])


def _make_sc_aggregate(D):
    @functools.partial(
        pl.kernel,
        out_type=jax.ShapeDtypeStruct((NC, NPAD, D), jnp.float32),
        mesh=_vector_mesh,
        scratch_types=[
            pltpu.VMEM((CHUNK, D), jnp.float32),
            pltpu.VMEM((CHUNK,), jnp.int32),
            pltpu.VMEM((CHUNK,), jnp.int32),
            pltpu.VMEM_SHARED((NPAD, D), jnp.float32),
            pltpu.SemaphoreType.DMA,
        ],
    )
    def _sc_aggregate(table_hbm, src_hbm, dst_hbm, out_hbm,
                      rows, idxs, idxd, acc, sem):
        cid = lax.axis_index("c")
        sid = lax.axis_index("s")
        wid = cid * NS + sid

        zeros16 = jnp.zeros((L,), jnp.float32)

        @pl.loop(0, CHUNK)
        def _(r):
            @pl.loop(0, D // L)
            def _(j):
                rows[r, pl.ds(j * L, L)] = zeros16

        base = sid * ROWS_PER_TILE
        for z in range(ROWS_PER_TILE // CHUNK):
            pltpu.sync_copy(rows, acc.at[pl.ds(base + z * CHUNK, CHUNK)])
        plsc.subcore_barrier()

        nfull = NCHUNKS // NW + (1 if NCHUNKS % NW else 0)

        @pl.loop(0, nfull)
        def _(c):
            g = c * NW + wid

            @pl.when(g < NCHUNKS)
            def _():
                pltpu.sync_copy(src_hbm.at[g], idxs)
                pltpu.sync_copy(dst_hbm.at[g], idxd)
                pltpu.async_copy(table_hbm.at[idxs], rows, sem).wait()
                pltpu.sync_copy(rows, acc.at[idxd], add=True)

        plsc.subcore_barrier()
        pltpu.sync_copy(acc.at[pl.ds(base, ROWS_PER_TILE)],
                        out_hbm.at[cid, pl.ds(base, ROWS_PER_TILE)])

    return _sc_aggregate


_sc_aggregate_h = _make_sc_aggregate(D_H)
_sc_aggregate_o = _make_sc_aggregate(D_OUT)


_BLK = 1000
_GRID = N // _BLK


def _dot(a, b):
    return lax.dot_general(a, b, (((1,), (0,)), ((), ())),
                           preferred_element_type=jnp.float32,
                           precision=lax.Precision.HIGHEST)


def _tc_matmul(x, w):
    k = w.shape[1]

    def body(x_ref, w_ref, o_ref):
        o_ref[...] = _dot(x_ref[...], w_ref[...])

    return pl.pallas_call(
        body,
        grid=(_GRID,),
        in_specs=[
            pl.BlockSpec((_BLK, x.shape[1]), lambda i: (i, 0)),
            pl.BlockSpec((x.shape[1], k), lambda i: (0, 0)),
        ],
        out_specs=pl.BlockSpec((_BLK, k), lambda i: (i, 0)),
        out_shape=jax.ShapeDtypeStruct((N, k), jnp.float32),
    )(x, w)


def _norm_from_parts(dp_ref):
    deg = dp_ref[0] + dp_ref[1]
    return lax.rsqrt(jnp.maximum(deg, 1.0))


def _tc_scale(xw, dego_parts):
    def body(x_ref, dp_ref, o_ref):
        o_ref[...] = x_ref[...] * _norm_from_parts(dp_ref)

    return pl.pallas_call(
        body,
        grid=(_GRID,),
        in_specs=[
            pl.BlockSpec((_BLK, D_H), lambda i: (i, 0)),
            pl.BlockSpec((NC, _BLK, 1), lambda i: (0, i, 0)),
        ],
        out_specs=pl.BlockSpec((_BLK, D_H), lambda i: (i, 0)),
        out_shape=jax.ShapeDtypeStruct((N, D_H), jnp.float32),
    )(xw, dego_parts)


def _tc_mid(parts, dego_parts, degi_parts, b1, w2):
    def body(p_ref, do_ref, di_ref, b_ref, w_ref, o_ref):
        agg = p_ref[0] + p_ref[1]
        h = jnp.maximum(agg * _norm_from_parts(di_ref) + b_ref[...], 0.0)
        o_ref[...] = _dot(h * _norm_from_parts(do_ref), w_ref[...])

    return pl.pallas_call(
        body,
        grid=(_GRID,),
        in_specs=[
            pl.BlockSpec((NC, _BLK, D_H), lambda i: (0, i, 0)),
            pl.BlockSpec((NC, _BLK, 1), lambda i: (0, i, 0)),
            pl.BlockSpec((NC, _BLK, 1), lambda i: (0, i, 0)),
            pl.BlockSpec((1, D_H), lambda i: (0, 0)),
            pl.BlockSpec((D_H, D_OUT), lambda i: (0, 0)),
        ],
        out_specs=pl.BlockSpec((_BLK, D_OUT), lambda i: (i, 0)),
        out_shape=jax.ShapeDtypeStruct((N, D_OUT), jnp.float32),
    )(parts, dego_parts, degi_parts, b1, w2)


def _tc_final(parts, degi_parts, b2):
    def body(p_ref, di_ref, b_ref, o_ref):
        agg = p_ref[0] + p_ref[1]
        o_ref[...] = agg * _norm_from_parts(di_ref) + b_ref[...]

    return pl.pallas_call(
        body,
        grid=(_GRID,),
        in_specs=[
            pl.BlockSpec((NC, _BLK, D_OUT), lambda i: (0, i, 0)),
            pl.BlockSpec((NC, _BLK, 1), lambda i: (0, i, 0)),
            pl.BlockSpec((1, D_OUT), lambda i: (0, 0)),
        ],
        out_specs=pl.BlockSpec((_BLK, D_OUT), lambda i: (i, 0)),
        out_shape=jax.ShapeDtypeStruct((N, D_OUT), jnp.float32),
    )(parts, degi_parts, b2)


def kernel(features, edge_index, W1, b1, W2, b2):
    src = edge_index[0].reshape(NCHUNKS, CHUNK)
    dst = edge_index[1].reshape(NCHUNKS, CHUNK)

    dego, degi = _sc_degrees(src, dst)
    xw1 = _tc_matmul(features, W1)

    dego3 = dego.reshape(NC, NPAD, 1)
    degi3 = degi.reshape(NC, NPAD, 1)

    table1 = _tc_scale(xw1, dego3)
    parts1 = _sc_aggregate_h(table1, src, dst)
    table2 = _tc_mid(parts1, dego3, degi3,
                     b1.reshape(1, D_H), W2)
    parts2 = _sc_aggregate_o(table2, src, dst)
    return _tc_final(parts2, degi3, b2.reshape(1, D_OUT))


# speedup vs baseline: 6.2483x; 6.2483x over previous
import dataclasses
import functools

import jax
import jax.numpy as jnp
from jax import lax
from jax.experimental import pallas as pl
from jax.experimental.pallas import tpu as pltpu
from jax.experimental.pallas import tpu_sc as plsc

N = 10000
E = 320000
D_IN = 128
D_H = 128
D_OUT = 64

NC = 2
NS = 16
L = 16
NW = NC * NS

CHUNK = 128
NCHUNKS = E // CHUNK
NPAD = 10240
ROWS_PER_TILE = NPAD // NS

_vector_mesh = plsc.VectorSubcoreMesh(core_axis_name="c", subcore_axis_name="s")

_sc_params = pltpu.CompilerParams()
if "needs_layout_passes" in pltpu.CompilerParams.__dataclass_fields__:
    _sc_params = dataclasses.replace(_sc_params, needs_layout_passes=False)
_sc_agg_params = dataclasses.replace(_sc_params, use_tc_tiling_on_sc=False)


@functools.partial(
    pl.kernel,
    out_type=(
        jax.ShapeDtypeStruct((NC, NPAD), jnp.float32),
        jax.ShapeDtypeStruct((NC, NPAD), jnp.float32),
    ),
    mesh=_vector_mesh,
    scratch_types=[
        pltpu.VMEM((NPAD,), jnp.float32),
        pltpu.VMEM((NPAD,), jnp.float32),
        pltpu.VMEM((CHUNK,), jnp.int32),
        pltpu.VMEM((ROWS_PER_TILE,), jnp.float32),
        pltpu.VMEM_SHARED((NS, NPAD), jnp.float32),
        pltpu.VMEM_SHARED((NS, NPAD), jnp.float32),
        pltpu.SemaphoreType.DMA,
    ],
    compiler_params=_sc_params,
)
def _sc_degrees(src_hbm, dst_hbm, dego_hbm, degi_hbm,
                ho, hi, idxb, tmp, sho, shi, sem):
    cid = lax.axis_index("c")
    sid = lax.axis_index("s")
    wid = cid * NS + sid

    zeros16 = jnp.zeros((L,), jnp.float32)
    ones16 = jnp.ones((L,), jnp.float32)

    @pl.loop(0, NPAD // L)
    def _(i):
        ho[pl.ds(i * L, L)] = zeros16
        hi[pl.ds(i * L, L)] = zeros16

    nfull = NCHUNKS // NW + (1 if NCHUNKS % NW else 0)

    @pl.loop(0, nfull)
    def _(c):
        g = c * NW + wid

        @pl.when(g < NCHUNKS)
        def _():
            pltpu.sync_copy(src_hbm.at[g], idxb)

            @pl.loop(0, CHUNK // L)
            def _(i):
                plsc.addupdate_scatter(ho, [idxb[pl.ds(i * L, L)]], ones16)

            pltpu.sync_copy(dst_hbm.at[g], idxb)

            @pl.loop(0, CHUNK // L)
            def _(i):
                plsc.addupdate_scatter(hi, [idxb[pl.ds(i * L, L)]], ones16)

    pltpu.sync_copy(ho, sho.at[sid])
    pltpu.sync_copy(hi, shi.at[sid])
    plsc.subcore_barrier()

    base = sid * ROWS_PER_TILE
    pltpu.sync_copy(sho.at[0, pl.ds(base, ROWS_PER_TILE)],
                    ho.at[pl.ds(0, ROWS_PER_TILE)])
    pltpu.sync_copy(shi.at[0, pl.ds(base, ROWS_PER_TILE)],
                    hi.at[pl.ds(0, ROWS_PER_TILE)])
    for t in range(1, NS):
        pltpu.sync_copy(sho.at[t, pl.ds(base, ROWS_PER_TILE)], tmp)

        @pl.loop(0, ROWS_PER_TILE // L)
        def _(j):
            ho[pl.ds(j * L, L)] = ho[pl.ds(j * L, L)] + tmp[pl.ds(j * L, L)]

        pltpu.sync_copy(shi.at[t, pl.ds(base, ROWS_PER_TILE)], tmp)

        @pl.loop(0, ROWS_PER_TILE // L)
        def _(j):
            hi[pl.ds(j * L, L)] = hi[pl.ds(j * L, L)] + tmp[pl.ds(j * L, L)]

    pltpu.sync_copy(ho.at[pl.ds(0, ROWS_PER_TILE)],
                    dego_hbm.at[cid, pl.ds(base, ROWS_PER_TILE)])
    pltpu.sync_copy(hi.at[pl.ds(0, ROWS_PER_TILE)],
                    degi_hbm.at[cid, pl.ds(base, ROWS_PER_TILE)])


def _make_sc_aggregate(D):
    @functools.partial(
        pl.kernel,
        out_type=jax.ShapeDtypeStruct((NC, NPAD, D), jnp.float32),
        mesh=_vector_mesh,
        scratch_types=[
            pltpu.VMEM((CHUNK, D), jnp.float32),
            pltpu.VMEM((CHUNK,), jnp.int32),
            pltpu.VMEM((CHUNK,), jnp.int32),
            pltpu.VMEM_SHARED((NPAD, D), jnp.float32),
            pltpu.SemaphoreType.DMA,
        ],
        compiler_params=_sc_agg_params,
    )
    def _sc_aggregate(table_hbm, src_hbm, dst_hbm, out_hbm,
                      rows, idxs, idxd, acc, sem):
        cid = lax.axis_index("c")
        sid = lax.axis_index("s")
        wid = cid * NS + sid

        zeros16 = jnp.zeros((L,), jnp.float32)

        @pl.loop(0, CHUNK)
        def _(r):
            @pl.loop(0, D // L)
            def _(j):
                rows[r, pl.ds(j * L, L)] = zeros16

        base = sid * ROWS_PER_TILE
        for z in range(ROWS_PER_TILE // CHUNK):
            pltpu.sync_copy(rows, acc.at[pl.ds(base + z * CHUNK, CHUNK)])
        plsc.subcore_barrier()

        nfull = NCHUNKS // NW + (1 if NCHUNKS % NW else 0)

        @pl.loop(0, nfull)
        def _(c):
            g = c * NW + wid

            @pl.when(g < NCHUNKS)
            def _():
                pltpu.sync_copy(src_hbm.at[g], idxs)
                pltpu.sync_copy(dst_hbm.at[g], idxd)
                pltpu.async_copy(table_hbm.at[idxs], rows, sem).wait()
                pltpu.sync_copy(rows, acc.at[idxd], add=True)

        plsc.subcore_barrier()
        pltpu.sync_copy(acc.at[pl.ds(base, ROWS_PER_TILE)],
                        out_hbm.at[cid, pl.ds(base, ROWS_PER_TILE)])

    return _sc_aggregate


_sc_aggregate_h = _make_sc_aggregate(D_H)
_sc_aggregate_o = _make_sc_aggregate(D_OUT)


_BLK = 1000
_GRID = N // _BLK


def _dot(a, b):
    return lax.dot_general(a, b, (((1,), (0,)), ((), ())),
                           preferred_element_type=jnp.float32,
                           precision=lax.Precision.HIGHEST)


def _tc_matmul(x, w):
    k = w.shape[1]

    def body(x_ref, w_ref, o_ref):
        o_ref[...] = _dot(x_ref[...], w_ref[...])

    return pl.pallas_call(
        body,
        grid=(_GRID,),
        in_specs=[
            pl.BlockSpec((_BLK, x.shape[1]), lambda i: (i, 0)),
            pl.BlockSpec((x.shape[1], k), lambda i: (0, 0)),
        ],
        out_specs=pl.BlockSpec((_BLK, k), lambda i: (i, 0)),
        out_shape=jax.ShapeDtypeStruct((N, k), jnp.float32),
    )(x, w)


def _norm_from_parts(dp_ref):
    deg = dp_ref[0] + dp_ref[1]
    return lax.rsqrt(jnp.maximum(deg, 1.0))


def _tc_scale(xw, dego_parts):
    def body(x_ref, dp_ref, o_ref):
        o_ref[...] = x_ref[...] * _norm_from_parts(dp_ref)

    return pl.pallas_call(
        body,
        grid=(_GRID,),
        in_specs=[
            pl.BlockSpec((_BLK, D_H), lambda i: (i, 0)),
            pl.BlockSpec((NC, _BLK, 1), lambda i: (0, i, 0)),
        ],
        out_specs=pl.BlockSpec((_BLK, D_H), lambda i: (i, 0)),
        out_shape=jax.ShapeDtypeStruct((N, D_H), jnp.float32),
    )(xw, dego_parts)


def _tc_mid(parts, dego_parts, degi_parts, b1, w2):
    def body(p_ref, do_ref, di_ref, b_ref, w_ref, o_ref):
        agg = p_ref[0] + p_ref[1]
        h = jnp.maximum(agg * _norm_from_parts(di_ref) + b_ref[...], 0.0)
        o_ref[...] = _dot(h * _norm_from_parts(do_ref), w_ref[...])

    return pl.pallas_call(
        body,
        grid=(_GRID,),
        in_specs=[
            pl.BlockSpec((NC, _BLK, D_H), lambda i: (0, i, 0)),
            pl.BlockSpec((NC, _BLK, 1), lambda i: (0, i, 0)),
            pl.BlockSpec((NC, _BLK, 1), lambda i: (0, i, 0)),
            pl.BlockSpec((1, D_H), lambda i: (0, 0)),
            pl.BlockSpec((D_H, D_OUT), lambda i: (0, 0)),
        ],
        out_specs=pl.BlockSpec((_BLK, D_OUT), lambda i: (i, 0)),
        out_shape=jax.ShapeDtypeStruct((N, D_OUT), jnp.float32),
    )(parts, dego_parts, degi_parts, b1, w2)


def _tc_final(parts, degi_parts, b2):
    def body(p_ref, di_ref, b_ref, o_ref):
        agg = p_ref[0] + p_ref[1]
        o_ref[...] = agg * _norm_from_parts(di_ref) + b_ref[...]

    return pl.pallas_call(
        body,
        grid=(_GRID,),
        in_specs=[
            pl.BlockSpec((NC, _BLK, D_OUT), lambda i: (0, i, 0)),
            pl.BlockSpec((NC, _BLK, 1), lambda i: (0, i, 0)),
            pl.BlockSpec((1, D_OUT), lambda i: (0, 0)),
        ],
        out_specs=pl.BlockSpec((_BLK, D_OUT), lambda i: (i, 0)),
        out_shape=jax.ShapeDtypeStruct((N, D_OUT), jnp.float32),
    )(parts, degi_parts, b2)


def kernel(features, edge_index, W1, b1, W2, b2):
    src = edge_index[0].reshape(NCHUNKS, CHUNK)
    dst = edge_index[1].reshape(NCHUNKS, CHUNK)

    dego, degi = _sc_degrees(src, dst)
    xw1 = _tc_matmul(features, W1)

    dego3 = dego.reshape(NC, NPAD, 1)
    degi3 = degi.reshape(NC, NPAD, 1)

    table1 = _tc_scale(xw1, dego3)
    parts1 = _sc_aggregate_h(table1, src, dst)
    table2 = _tc_mid(parts1, dego3, degi3,
                     b1.reshape(1, D_H), W2)
    parts2 = _sc_aggregate_o(table2, src, dst)
    return _tc_final(parts2, degi3, b2.reshape(1, D_OUT))


# speedup vs baseline: 10.8295x; 1.7332x over previous
import dataclasses
import functools

import jax
import jax.numpy as jnp
from jax import lax
from jax.experimental import pallas as pl
from jax.experimental.pallas import tpu as pltpu
from jax.experimental.pallas import tpu_sc as plsc

N = 10000
E = 320000
D_IN = 128
D_H = 128
D_OUT = 64

NC = 2
NS = 16
L = 16
NW = NC * NS

CHUNK = 128
NCHUNKS = E // CHUNK
NPAD = 10240
ROWS_PER_TILE = NPAD // NS

_vector_mesh = plsc.VectorSubcoreMesh(core_axis_name="c", subcore_axis_name="s")

_sc_params = pltpu.CompilerParams()
if "needs_layout_passes" in pltpu.CompilerParams.__dataclass_fields__:
    _sc_params = dataclasses.replace(_sc_params, needs_layout_passes=False)
_sc_agg_params = dataclasses.replace(_sc_params, use_tc_tiling_on_sc=False)


EPT = E // NW
NPHASE = 3
PH = 4096
NPAD_DEG = NPHASE * PH
RPP = PH // NS


@functools.partial(
    pl.kernel,
    out_type=(
        jax.ShapeDtypeStruct((NC, NPAD_DEG), jnp.float32),
        jax.ShapeDtypeStruct((NC, NPAD_DEG), jnp.float32),
    ),
    mesh=_vector_mesh,
    scratch_types=[
        pltpu.VMEM((NPAD_DEG,), jnp.float32),
        pltpu.VMEM((NPAD_DEG,), jnp.float32),
        pltpu.VMEM((EPT,), jnp.int32),
        pltpu.VMEM((NS, RPP), jnp.float32),
        pltpu.VMEM((RPP,), jnp.float32),
        pltpu.VMEM_SHARED((NS, PH), jnp.float32),
        pltpu.SemaphoreType.DMA,
    ],
    compiler_params=_sc_params,
)
def _sc_degrees(src_hbm, dst_hbm, dego_hbm, degi_hbm,
                ho, hi, idxb, tmp2d, red, sh, sem):
    cid = lax.axis_index("c")
    sid = lax.axis_index("s")
    wid = cid * NS + sid

    zeros16 = jnp.zeros((L,), jnp.float32)
    ones16 = jnp.ones((L,), jnp.float32)

    @pl.loop(0, NPAD_DEG // L, unroll=8)
    def _(i):
        ho[pl.ds(i * L, L)] = zeros16
        hi[pl.ds(i * L, L)] = zeros16

    pltpu.sync_copy(src_hbm.at[pl.ds(wid * EPT, EPT)], idxb)

    @pl.loop(0, EPT // L, unroll=8)
    def _(i):
        plsc.addupdate_scatter(ho, [idxb[pl.ds(i * L, L)]], ones16)

    pltpu.sync_copy(dst_hbm.at[pl.ds(wid * EPT, EPT)], idxb)

    @pl.loop(0, EPT // L, unroll=8)
    def _(i):
        plsc.addupdate_scatter(hi, [idxb[pl.ds(i * L, L)]], ones16)

    base = sid * RPP
    for hist, out in ((ho, dego_hbm), (hi, degi_hbm)):
        for h in range(NPHASE):
            pltpu.sync_copy(hist.at[pl.ds(h * PH, PH)], sh.at[sid])
            plsc.subcore_barrier()
            pltpu.sync_copy(sh.at[:, pl.ds(base, RPP)], tmp2d)

            @pl.loop(0, RPP // L, unroll=4)
            def _(j):
                s = pl.ds(j * L, L)
                acc = tmp2d[0, s]
                for t in range(1, NS):
                    acc = acc + tmp2d[t, s]
                red[s] = acc

            pltpu.sync_copy(red, out.at[cid, pl.ds(h * PH + base, RPP)])
            plsc.subcore_barrier()


CPT = NCHUNKS // NS
CPT_REM = NCHUNKS % NS
CPT_MAX = CPT + 1
NCHUNKS_PAD = 2504


def _make_sc_aggregate(DH):
    @functools.partial(
        pl.kernel,
        out_type=jax.ShapeDtypeStruct((NC, NPAD, DH), jnp.float32),
        mesh=_vector_mesh,
        scratch_types=[
            pltpu.VMEM((CHUNK, DH), jnp.float32),
            pltpu.VMEM((CHUNK, DH), jnp.float32),
            pltpu.VMEM((CPT_MAX, CHUNK), jnp.int32),
            pltpu.VMEM((CPT_MAX, CHUNK), jnp.int32),
            pltpu.VMEM_SHARED((NPAD, DH), jnp.float32),
            pltpu.SemaphoreType.DMA,
            pltpu.SemaphoreType.DMA,
        ],
        compiler_params=_sc_agg_params,
    )
    def _sc_aggregate(table_hbm, src_hbm, dst_hbm, out_hbm,
                      rows_a, rows_b, idxs, idxd, acc, sem_a, sem_b):
        cid = lax.axis_index("c")
        sid = lax.axis_index("s")

        zeros16 = jnp.zeros((L,), jnp.float32)

        start = CPT * sid + jnp.minimum(sid, CPT_REM)
        ncw = CPT + jnp.where(sid < CPT_REM, 1, 0)

        pltpu.sync_copy(src_hbm.at[cid, pl.ds(start, CPT_MAX)], idxs)
        pltpu.sync_copy(dst_hbm.at[pl.ds(start, CPT_MAX)], idxd)

        @pl.loop(0, CHUNK)
        def _(r):
            @pl.loop(0, DH // L)
            def _(j):
                rows_a[r, pl.ds(j * L, L)] = zeros16

        base = sid * ROWS_PER_TILE
        for z in range(ROWS_PER_TILE // CHUNK):
            pltpu.sync_copy(rows_a, acc.at[pl.ds(base + z * CHUNK, CHUNK)])
        plsc.subcore_barrier()

        def gather(c, rows, sem):
            return pltpu.async_copy(table_hbm.at[idxs.at[c]], rows, sem)

        def scatter_add(c, rows):
            pltpu.sync_copy(rows, acc.at[idxd.at[c]], add=True)

        gather(0, rows_a, sem_a)
        gather(1, rows_b, sem_b)

        @pl.loop(0, CPT // 2)
        def _(p):
            c = 2 * p
            pltpu.make_async_copy(table_hbm.at[idxs.at[c]], rows_a, sem_a).wait()
            scatter_add(c, rows_a)

            @pl.when(c + 2 < CPT)
            def _():
                gather(c + 2, rows_a, sem_a)

            pltpu.make_async_copy(table_hbm.at[idxs.at[c]], rows_b, sem_b).wait()
            scatter_add(c + 1, rows_b)

            @pl.when(c + 3 < CPT)
            def _():
                gather(c + 3, rows_b, sem_b)

        @pl.when(ncw > CPT)
        def _():
            gather(CPT, rows_a, sem_a).wait()
            scatter_add(CPT, rows_a)

        plsc.subcore_barrier()
        pltpu.sync_copy(acc.at[pl.ds(base, ROWS_PER_TILE)],
                        out_hbm.at[cid, pl.ds(base, ROWS_PER_TILE)])

    return _sc_aggregate


_sc_aggregate_h = _make_sc_aggregate(D_H // 2)
_sc_aggregate_o = _make_sc_aggregate(D_OUT // 2)


_BLK = 1000
_GRID = N // _BLK


def _dot(a, b):
    return lax.dot_general(a, b, (((1,), (0,)), ((), ())),
                           preferred_element_type=jnp.float32,
                           precision=lax.Precision.HIGHEST)


def _tc_matmul(x, w):
    k = w.shape[1]

    def body(x_ref, w_ref, o_ref):
        o_ref[...] = _dot(x_ref[...], w_ref[...])

    return pl.pallas_call(
        body,
        grid=(_GRID,),
        in_specs=[
            pl.BlockSpec((_BLK, x.shape[1]), lambda i: (i, 0)),
            pl.BlockSpec((x.shape[1], k), lambda i: (0, 0)),
        ],
        out_specs=pl.BlockSpec((_BLK, k), lambda i: (i, 0)),
        out_shape=jax.ShapeDtypeStruct((N, k), jnp.float32),
    )(x, w)


def _norm_from_parts(dp_ref):
    deg = dp_ref[0] + dp_ref[1]
    return lax.rsqrt(jnp.maximum(deg, 1.0))


def _tc_scale(xw, dego_parts):
    DH = D_H // 2

    def body(x_ref, dp_ref, o_ref):
        nsrc = _norm_from_parts(dp_ref)
        o_ref[0] = x_ref[:, :DH] * nsrc
        o_ref[1] = x_ref[:, DH:] * nsrc

    return pl.pallas_call(
        body,
        grid=(_GRID,),
        in_specs=[
            pl.BlockSpec((_BLK, D_H), lambda i: (i, 0)),
            pl.BlockSpec((NC, _BLK, 1), lambda i: (0, i, 0)),
        ],
        out_specs=pl.BlockSpec((2, _BLK, DH), lambda i: (0, i, 0)),
        out_shape=jax.ShapeDtypeStruct((2, N, DH), jnp.float32),
    )(xw, dego_parts)


def _tc_mid(parts, dego_parts, degi_parts, b1, w2):
    DO = D_OUT // 2

    def body(p_ref, do_ref, di_ref, b_ref, w_ref, o_ref):
        agg = jnp.concatenate([p_ref[0], p_ref[1]], axis=-1)
        h = jnp.maximum(agg * _norm_from_parts(di_ref) + b_ref[...], 0.0)
        hn = h * _norm_from_parts(do_ref)
        o_ref[0] = _dot(hn, w_ref[:, :DO])
        o_ref[1] = _dot(hn, w_ref[:, DO:])

    return pl.pallas_call(
        body,
        grid=(_GRID,),
        in_specs=[
            pl.BlockSpec((NC, _BLK, D_H // 2), lambda i: (0, i, 0)),
            pl.BlockSpec((NC, _BLK, 1), lambda i: (0, i, 0)),
            pl.BlockSpec((NC, _BLK, 1), lambda i: (0, i, 0)),
            pl.BlockSpec((1, D_H), lambda i: (0, 0)),
            pl.BlockSpec((D_H, D_OUT), lambda i: (0, 0)),
        ],
        out_specs=pl.BlockSpec((2, _BLK, DO), lambda i: (0, i, 0)),
        out_shape=jax.ShapeDtypeStruct((2, N, DO), jnp.float32),
    )(parts, dego_parts, degi_parts, b1, w2)


def _tc_final(parts, degi_parts, b2):
    def body(p_ref, di_ref, b_ref, o_ref):
        agg = jnp.concatenate([p_ref[0], p_ref[1]], axis=-1)
        o_ref[...] = agg * _norm_from_parts(di_ref) + b_ref[...]

    return pl.pallas_call(
        body,
        grid=(_GRID,),
        in_specs=[
            pl.BlockSpec((NC, _BLK, D_OUT // 2), lambda i: (0, i, 0)),
            pl.BlockSpec((NC, _BLK, 1), lambda i: (0, i, 0)),
            pl.BlockSpec((1, D_OUT), lambda i: (0, 0)),
        ],
        out_specs=pl.BlockSpec((_BLK, D_OUT), lambda i: (i, 0)),
        out_shape=jax.ShapeDtypeStruct((N, D_OUT), jnp.float32),
    )(parts, degi_parts, b2)


def kernel(features, edge_index, W1, b1, W2, b2):
    pad = ((0, NCHUNKS_PAD - NCHUNKS), (0, 0))
    src0 = edge_index[0].reshape(NCHUNKS, CHUNK)
    src = jnp.pad(jnp.stack([src0, src0 + N]), ((0, 0),) + pad)
    dst = jnp.pad(edge_index[1].reshape(NCHUNKS, CHUNK), pad)

    dego, degi = _sc_degrees(edge_index[0], edge_index[1])
    xw1 = _tc_matmul(features, W1)

    dego3 = dego.reshape(NC, NPAD_DEG, 1)
    degi3 = degi.reshape(NC, NPAD_DEG, 1)

    table1 = _tc_scale(xw1, dego3).reshape(2 * N, D_H // 2)
    parts1 = _sc_aggregate_h(table1, src, dst)
    table2 = _tc_mid(parts1, dego3, degi3,
                     b1.reshape(1, D_H), W2).reshape(2 * N, D_OUT // 2)
    parts2 = _sc_aggregate_o(table2, src, dst)
    return _tc_final(parts2, degi3, b2.reshape(1, D_OUT))


# speedup vs baseline: 12.9514x; 1.1959x over previous
import dataclasses
import functools

import jax
import jax.numpy as jnp
from jax import lax
from jax.experimental import pallas as pl
from jax.experimental.pallas import tpu as pltpu
from jax.experimental.pallas import tpu_sc as plsc

N = 10000
E = 320000
D_IN = 128
D_H = 128
D_OUT = 64

NC = 2
NS = 16
L = 16
NW = NC * NS

CHUNK = 128
NCHUNKS = E // CHUNK
NPAD = 10240
ROWS_PER_TILE = NPAD // NS

_vector_mesh = plsc.VectorSubcoreMesh(core_axis_name="c", subcore_axis_name="s")

_sc_params = pltpu.CompilerParams()
if "needs_layout_passes" in pltpu.CompilerParams.__dataclass_fields__:
    _sc_params = dataclasses.replace(_sc_params, needs_layout_passes=False)
_sc_agg_params = dataclasses.replace(_sc_params, use_tc_tiling_on_sc=False,
                                     internal_scratch_in_bytes=262144)


EPT = E // NW
NPAD_DEG = 10240


@functools.partial(
    pl.kernel,
    out_type=(
        jax.ShapeDtypeStruct((NW, NPAD_DEG), jnp.float32),
        jax.ShapeDtypeStruct((NW, NPAD_DEG), jnp.float32),
    ),
    mesh=_vector_mesh,
    scratch_types=[
        pltpu.VMEM((NPAD_DEG,), jnp.float32),
        pltpu.VMEM((NPAD_DEG,), jnp.float32),
        pltpu.VMEM((EPT,), jnp.int32),
        pltpu.SemaphoreType.DMA,
    ],
    compiler_params=_sc_params,
)
def _sc_degrees(ei_hbm, dego_hbm, degi_hbm, ho, hi, idxb, sem):
    cid = lax.axis_index("c")
    sid = lax.axis_index("s")
    wid = cid * NS + sid

    zeros16 = jnp.zeros((L,), jnp.float32)
    ones16 = jnp.ones((L,), jnp.float32)

    @pl.loop(0, NPAD_DEG // L, unroll=8)
    def _(i):
        ho[pl.ds(i * L, L)] = zeros16
        hi[pl.ds(i * L, L)] = zeros16

    pltpu.sync_copy(ei_hbm.at[pl.ds(wid * EPT, EPT)], idxb)

    @pl.loop(0, EPT // L, unroll=8)
    def _(i):
        plsc.addupdate_scatter(ho, [idxb[pl.ds(i * L, L)]], ones16)

    pltpu.sync_copy(ei_hbm.at[pl.ds(E + wid * EPT, EPT)], idxb)

    @pl.loop(0, EPT // L, unroll=8)
    def _(i):
        plsc.addupdate_scatter(hi, [idxb[pl.ds(i * L, L)]], ones16)

    pltpu.sync_copy(ho, dego_hbm.at[wid])
    pltpu.sync_copy(hi, degi_hbm.at[wid])


def _make_sc_aggregate(D, col_split):
    nsplit = NS if col_split else NW
    cpt = NCHUNKS // nsplit
    rem = NCHUNKS % nsplit
    assert cpt % 2 == 0 and rem % 2 == 0
    nxt = rem // 2

    @functools.partial(
        pl.kernel,
        out_type=jax.ShapeDtypeStruct((NC, NPAD, D), jnp.float32),
        mesh=_vector_mesh,
        scratch_types=[
            pltpu.VMEM((CHUNK, D), jnp.float32),
            pltpu.VMEM((CHUNK, D), jnp.float32),
            pltpu.VMEM((cpt + 2, CHUNK), jnp.int32),
            pltpu.VMEM((cpt + 2, CHUNK), jnp.int32),
            pltpu.VMEM_SHARED((NPAD, D), jnp.float32),
            pltpu.SemaphoreType.DMA,
            pltpu.SemaphoreType.DMA,
        ],
        compiler_params=_sc_agg_params,
    )
    def _sc_aggregate(table_hbm, ei_hbm, out_hbm,
                      rows_a, rows_b, idxs, idxd, acc, sem_a, sem_b):
        cid = lax.axis_index("c")
        sid = lax.axis_index("s")
        w = sid if col_split else cid * NS + sid

        zeros16 = jnp.zeros((L,), jnp.float32)

        start = cpt * w + 2 * jnp.minimum(w, nxt)
        has_extra = w < nxt

        pltpu.sync_copy(ei_hbm.at[0, pl.ds(start, cpt)],
                        idxs.at[pl.ds(0, cpt)])
        pltpu.sync_copy(ei_hbm.at[1, pl.ds(start, cpt)],
                        idxd.at[pl.ds(0, cpt)])

        @pl.when(has_extra)
        def _():
            pltpu.sync_copy(ei_hbm.at[0, pl.ds(start + cpt, 2)],
                            idxs.at[pl.ds(cpt, 2)])
            pltpu.sync_copy(ei_hbm.at[1, pl.ds(start + cpt, 2)],
                            idxd.at[pl.ds(cpt, 2)])

        if col_split:
            off = jnp.full((L,), 1, jnp.int32) * cid

            @pl.loop(0, cpt + 2)
            def _(r):
                @pl.loop(0, CHUNK // L, unroll=8)
                def _(j):
                    s = pl.ds(j * L, L)
                    idxs[r, s] = idxs[r, s] + idxs[r, s] + off

        @pl.loop(0, CHUNK)
        def _(r):
            @pl.loop(0, D // L)
            def _(j):
                rows_a[r, pl.ds(j * L, L)] = zeros16

        base = sid * ROWS_PER_TILE
        for z in range(ROWS_PER_TILE // CHUNK):
            pltpu.sync_copy(rows_a, acc.at[pl.ds(base + z * CHUNK, CHUNK)])
        plsc.subcore_barrier()

        def gather(c, rows, sem):
            return pltpu.async_copy(table_hbm.at[idxs.at[c]], rows, sem)

        def scatter_add(c, rows):
            pltpu.sync_copy(rows, acc.at[idxd.at[c]], add=True)

        gather(0, rows_a, sem_a)
        gather(1, rows_b, sem_b)

        @pl.loop(0, cpt // 2)
        def _(p):
            c = 2 * p
            pltpu.make_async_copy(table_hbm.at[idxs.at[c]], rows_a, sem_a).wait()
            scatter_add(c, rows_a)

            @pl.when(c + 2 < cpt)
            def _():
                gather(c + 2, rows_a, sem_a)

            pltpu.make_async_copy(table_hbm.at[idxs.at[c]], rows_b, sem_b).wait()
            scatter_add(c + 1, rows_b)

            @pl.when(c + 3 < cpt)
            def _():
                gather(c + 3, rows_b, sem_b)

        @pl.when(has_extra)
        def _():
            da = gather(cpt, rows_a, sem_a)
            db = gather(cpt + 1, rows_b, sem_b)
            da.wait()
            scatter_add(cpt, rows_a)
            db.wait()
            scatter_add(cpt + 1, rows_b)

        plsc.subcore_barrier()
        pltpu.sync_copy(acc.at[pl.ds(base, ROWS_PER_TILE)],
                        out_hbm.at[cid, pl.ds(base, ROWS_PER_TILE)])

    return _sc_aggregate


_sc_aggregate_h = _make_sc_aggregate(D_H // 2, col_split=True)
_sc_aggregate_o = _make_sc_aggregate(D_OUT, col_split=False)


_BLK = 1000
_GRID = N // _BLK


def _dot(a, b):
    return lax.dot_general(a, b, (((1,), (0,)), ((), ())),
                           preferred_element_type=jnp.float32,
                           precision=lax.Precision.HIGHEST)


def _tc_matmul(x, w):
    k = w.shape[1]

    def body(x_ref, w_ref, o_ref):
        o_ref[...] = _dot(x_ref[...], w_ref[...])

    return pl.pallas_call(
        body,
        grid=(_GRID,),
        in_specs=[
            pl.BlockSpec((_BLK, x.shape[1]), lambda i: (i, 0)),
            pl.BlockSpec((x.shape[1], k), lambda i: (0, 0)),
        ],
        out_specs=pl.BlockSpec((_BLK, k), lambda i: (i, 0)),
        out_shape=jax.ShapeDtypeStruct((N, k), jnp.float32),
    )(x, w)


def _tc_norms(dego_raw, degi_raw):
    blk = 1024
    grid = NPAD_DEG // blk

    def body(do_ref, di_ref, no_ref, ni_ref):
        for d_ref, n_ref in ((do_ref, no_ref), (di_ref, ni_ref)):
            deg = jnp.sum(d_ref[...], axis=0, keepdims=True)
            norm = lax.rsqrt(jnp.maximum(deg, 1.0))
            n_ref[...] = jnp.transpose(norm, (1, 0))

    return pl.pallas_call(
        body,
        grid=(grid,),
        in_specs=[
            pl.BlockSpec((NW, blk), lambda i: (0, i)),
            pl.BlockSpec((NW, blk), lambda i: (0, i)),
        ],
        out_specs=[
            pl.BlockSpec((blk, 1), lambda i: (i, 0)),
            pl.BlockSpec((blk, 1), lambda i: (i, 0)),
        ],
        out_shape=[
            jax.ShapeDtypeStruct((NPAD_DEG, 1), jnp.float32),
            jax.ShapeDtypeStruct((NPAD_DEG, 1), jnp.float32),
        ],
    )(dego_raw, degi_raw)


def _tc_scale(xw, nsrc):
    def body(x_ref, n_ref, o_ref):
        o_ref[...] = x_ref[...] * n_ref[...]

    return pl.pallas_call(
        body,
        grid=(_GRID,),
        in_specs=[
            pl.BlockSpec((_BLK, D_H), lambda i: (i, 0)),
            pl.BlockSpec((_BLK, 1), lambda i: (i, 0)),
        ],
        out_specs=pl.BlockSpec((_BLK, D_H), lambda i: (i, 0)),
        out_shape=jax.ShapeDtypeStruct((N, D_H), jnp.float32),
    )(xw, nsrc)


def _tc_mid(parts, nsrc, ndst, b1, w2):
    def body(p_ref, ns_ref, nd_ref, b_ref, w_ref, o_ref):
        agg = jnp.concatenate([p_ref[0], p_ref[1]], axis=-1)
        h = jnp.maximum(agg * nd_ref[...] + b_ref[...], 0.0)
        o_ref[...] = _dot(h * ns_ref[...], w_ref[...])

    return pl.pallas_call(
        body,
        grid=(_GRID,),
        in_specs=[
            pl.BlockSpec((NC, _BLK, D_H // 2), lambda i: (0, i, 0)),
            pl.BlockSpec((_BLK, 1), lambda i: (i, 0)),
            pl.BlockSpec((_BLK, 1), lambda i: (i, 0)),
            pl.BlockSpec((1, D_H), lambda i: (0, 0)),
            pl.BlockSpec((D_H, D_OUT), lambda i: (0, 0)),
        ],
        out_specs=pl.BlockSpec((_BLK, D_OUT), lambda i: (i, 0)),
        out_shape=jax.ShapeDtypeStruct((N, D_OUT), jnp.float32),
    )(parts, nsrc, ndst, b1, w2)


def _tc_final(parts, ndst, b2):
    def body(p_ref, nd_ref, b_ref, o_ref):
        agg = p_ref[0] + p_ref[1]
        o_ref[...] = agg * nd_ref[...] + b_ref[...]

    return pl.pallas_call(
        body,
        grid=(_GRID,),
        in_specs=[
            pl.BlockSpec((NC, _BLK, D_OUT), lambda i: (0, i, 0)),
            pl.BlockSpec((_BLK, 1), lambda i: (i, 0)),
            pl.BlockSpec((1, D_OUT), lambda i: (0, 0)),
        ],
        out_specs=pl.BlockSpec((_BLK, D_OUT), lambda i: (i, 0)),
        out_shape=jax.ShapeDtypeStruct((N, D_OUT), jnp.float32),
    )(parts, ndst, b2)


def kernel(features, edge_index, W1, b1, W2, b2):
    ei = edge_index.reshape(2, NCHUNKS, CHUNK)

    dego, degi = _sc_degrees(edge_index.reshape(2 * E))
    xw1 = _tc_matmul(features, W1)
    nsrc, ndst = _tc_norms(dego, degi)

    table1 = _tc_scale(xw1, nsrc)
    parts1 = _sc_aggregate_h(table1.reshape(2 * N, D_H // 2), ei)
    table2 = _tc_mid(parts1, nsrc, ndst,
                     b1.reshape(1, D_H), W2)
    parts2 = _sc_aggregate_o(table2, ei)
    return _tc_final(parts2, ndst, b2.reshape(1, D_OUT))


# speedup vs baseline: 14.5567x; 1.1239x over previous
import dataclasses
import functools

import jax
import jax.numpy as jnp
from jax import lax
from jax.experimental import pallas as pl
from jax.experimental.pallas import tpu as pltpu
from jax.experimental.pallas import tpu_sc as plsc

N = 10000
E = 320000
D_IN = 128
D_H = 128
D_OUT = 64

NC = 2
NS = 16
L = 16
NW = NC * NS

CHUNK = 128
NCHUNKS = E // CHUNK
NPAD = 10240
ROWS_PER_TILE = NPAD // NS

_vector_mesh = plsc.VectorSubcoreMesh(core_axis_name="c", subcore_axis_name="s")

_sc_params = pltpu.CompilerParams()
if "needs_layout_passes" in pltpu.CompilerParams.__dataclass_fields__:
    _sc_params = dataclasses.replace(_sc_params, needs_layout_passes=False)
_sc_agg_params = dataclasses.replace(_sc_params, use_tc_tiling_on_sc=False,
                                     internal_scratch_in_bytes=262144)


NPAD_DEG = 10240
DCPT = NCHUNKS // NW
DREM = (NCHUNKS % NW) // 2


@functools.partial(
    pl.kernel,
    out_type=(
        jax.ShapeDtypeStruct((NW, NPAD_DEG), jnp.float32),
        jax.ShapeDtypeStruct((NW, NPAD_DEG), jnp.float32),
    ),
    mesh=_vector_mesh,
    scratch_types=[
        pltpu.VMEM((NPAD_DEG,), jnp.float32),
        pltpu.VMEM((NPAD_DEG,), jnp.float32),
        pltpu.VMEM((DCPT + 2, CHUNK), jnp.int32),
        pltpu.SemaphoreType.DMA,
    ],
    compiler_params=_sc_agg_params,
)
def _sc_degrees(ei_hbm, dego_hbm, degi_hbm, ho, hi, idxb, sem):
    cid = lax.axis_index("c")
    sid = lax.axis_index("s")
    wid = cid * NS + sid

    zeros16 = jnp.zeros((L,), jnp.float32)
    ones16 = jnp.ones((L,), jnp.float32)

    @pl.loop(0, NPAD_DEG // L, unroll=8)
    def _(i):
        ho[pl.ds(i * L, L)] = zeros16
        hi[pl.ds(i * L, L)] = zeros16

    start = DCPT * wid + 2 * jnp.minimum(wid, DREM)
    ncw = DCPT + jnp.where(wid < DREM, 2, 0)

    for e, hist in ((0, ho), (1, hi)):
        pltpu.sync_copy(ei_hbm.at[e, pl.ds(start, DCPT)],
                        idxb.at[pl.ds(0, DCPT)])

        @pl.when(ncw > DCPT)
        def _():
            pltpu.sync_copy(ei_hbm.at[e, pl.ds(start + DCPT, 2)],
                            idxb.at[pl.ds(DCPT, 2)])

        @pl.loop(0, ncw)
        def _(r):
            @pl.loop(0, CHUNK // L, unroll=8)
            def _(i):
                plsc.addupdate_scatter(hist, [idxb[r, pl.ds(i * L, L)]], ones16)

    pltpu.sync_copy(ho, dego_hbm.at[wid])
    pltpu.sync_copy(hi, degi_hbm.at[wid])


def _make_sc_aggregate(D, col_split):
    nsplit = NS if col_split else NW
    cpt = NCHUNKS // nsplit
    rem = NCHUNKS % nsplit
    assert cpt % 2 == 0 and rem % 2 == 0
    nxt = rem // 2

    @functools.partial(
        pl.kernel,
        out_type=jax.ShapeDtypeStruct((NC, NPAD, D), jnp.float32),
        mesh=_vector_mesh,
        scratch_types=[
            pltpu.VMEM((CHUNK, D), jnp.float32),
            pltpu.VMEM((CHUNK, D), jnp.float32),
            pltpu.VMEM((CHUNK, D), jnp.float32),
            pltpu.VMEM((CHUNK, D), jnp.float32),
            pltpu.VMEM((cpt + 2, CHUNK), jnp.int32),
            pltpu.VMEM((cpt + 2, CHUNK), jnp.int32),
            pltpu.VMEM_SHARED((NPAD, D), jnp.float32),
            pltpu.SemaphoreType.DMA,
            pltpu.SemaphoreType.DMA,
            pltpu.SemaphoreType.DMA,
            pltpu.SemaphoreType.DMA,
        ],
        compiler_params=_sc_agg_params,
    )
    def _sc_aggregate(table_hbm, ei_hbm, out_hbm,
                      r0, r1, r2, r3, idxs, idxd, acc, s0, s1, s2, s3):
        cid = lax.axis_index("c")
        sid = lax.axis_index("s")
        w = sid if col_split else cid * NS + sid

        bufs = (r0, r1, r2, r3)
        sems = (s0, s1, s2, s3)

        zeros16 = jnp.zeros((L,), jnp.float32)

        start = cpt * w + 2 * jnp.minimum(w, nxt)
        has_extra = w < nxt

        pltpu.sync_copy(ei_hbm.at[0, pl.ds(start, cpt)],
                        idxs.at[pl.ds(0, cpt)])
        pltpu.sync_copy(ei_hbm.at[1, pl.ds(start, cpt)],
                        idxd.at[pl.ds(0, cpt)])

        @pl.when(has_extra)
        def _():
            pltpu.sync_copy(ei_hbm.at[0, pl.ds(start + cpt, 2)],
                            idxs.at[pl.ds(cpt, 2)])
            pltpu.sync_copy(ei_hbm.at[1, pl.ds(start + cpt, 2)],
                            idxd.at[pl.ds(cpt, 2)])

        if col_split:
            off = jnp.full((L,), 1, jnp.int32) * cid

            @pl.loop(0, cpt + 2)
            def _(r):
                @pl.loop(0, CHUNK // L, unroll=8)
                def _(j):
                    s = pl.ds(j * L, L)
                    idxs[r, s] = idxs[r, s] + idxs[r, s] + off

        @pl.loop(0, CHUNK)
        def _(r):
            @pl.loop(0, D // L)
            def _(j):
                r0[r, pl.ds(j * L, L)] = zeros16

        base = sid * ROWS_PER_TILE
        for z in range(ROWS_PER_TILE // CHUNK):
            pltpu.sync_copy(r0, acc.at[pl.ds(base + z * CHUNK, CHUNK)])
        plsc.subcore_barrier()

        def gather(c, k, sem=None):
            return pltpu.async_copy(table_hbm.at[idxs.at[c]], bufs[k],
                                    sems[k] if sem is None else sem)

        def wait_gather(c, k):
            pltpu.make_async_copy(table_hbm.at[idxs.at[c]], bufs[k],
                                  sems[k]).wait()

        def scatter_add(c, k):
            pltpu.sync_copy(bufs[k], acc.at[idxd.at[c]], add=True)

        gather(0, 0)
        gather(1, 1)
        gather(2, 2)

        nloop = cpt // 4
        tail = cpt % 4

        @pl.loop(0, nloop)
        def _(p):
            c = 4 * p
            for k in range(4):
                wait_gather(c + k, k)
                scatter_add(c + k, k)

                @pl.when(c + k + 3 < cpt)
                def _():
                    gather(c + k + 3, (k + 3) % 4)

        cbase = nloop * 4
        for t in range(tail):
            c = cbase + t
            k = (cbase + t) % 4
            wait_gather(c, k)
            scatter_add(c, k)

        @pl.when(has_extra)
        def _():
            da = gather(cpt, 0)
            db = gather(cpt + 1, 1)
            da.wait()
            scatter_add(cpt, 0)
            db.wait()
            scatter_add(cpt + 1, 1)

        plsc.subcore_barrier()
        pltpu.sync_copy(acc.at[pl.ds(base, ROWS_PER_TILE)],
                        out_hbm.at[cid, pl.ds(base, ROWS_PER_TILE)])

    return _sc_aggregate


_sc_aggregate_h = _make_sc_aggregate(D_H // 2, col_split=True)
_sc_aggregate_o = _make_sc_aggregate(D_OUT, col_split=False)


_BLK = 1000
_GRID = N // _BLK


def _dot(a, b):
    return lax.dot_general(a, b, (((1,), (0,)), ((), ())),
                           preferred_element_type=jnp.float32,
                           precision=lax.Precision.HIGHEST)


def _tc_matmul(x, w):
    k = w.shape[1]

    def body(x_ref, w_ref, o_ref):
        o_ref[...] = _dot(x_ref[...], w_ref[...])

    return pl.pallas_call(
        body,
        grid=(_GRID,),
        in_specs=[
            pl.BlockSpec((_BLK, x.shape[1]), lambda i: (i, 0)),
            pl.BlockSpec((x.shape[1], k), lambda i: (0, 0)),
        ],
        out_specs=pl.BlockSpec((_BLK, k), lambda i: (i, 0)),
        out_shape=jax.ShapeDtypeStruct((N, k), jnp.float32),
    )(x, w)


def _tc_norms(dego_raw, degi_raw):
    blk = 1024
    grid = NPAD_DEG // blk

    def body(do_ref, di_ref, no_ref, ni_ref):
        for d_ref, n_ref in ((do_ref, no_ref), (di_ref, ni_ref)):
            deg = jnp.sum(d_ref[...], axis=0, keepdims=True)
            norm = lax.rsqrt(jnp.maximum(deg, 1.0))
            n_ref[...] = jnp.transpose(norm, (1, 0))

    return pl.pallas_call(
        body,
        grid=(grid,),
        in_specs=[
            pl.BlockSpec((NW, blk), lambda i: (0, i)),
            pl.BlockSpec((NW, blk), lambda i: (0, i)),
        ],
        out_specs=[
            pl.BlockSpec((blk, 1), lambda i: (i, 0)),
            pl.BlockSpec((blk, 1), lambda i: (i, 0)),
        ],
        out_shape=[
            jax.ShapeDtypeStruct((NPAD_DEG, 1), jnp.float32),
            jax.ShapeDtypeStruct((NPAD_DEG, 1), jnp.float32),
        ],
    )(dego_raw, degi_raw)


def _tc_scale(xw, nsrc):
    def body(x_ref, n_ref, o_ref):
        o_ref[...] = x_ref[...] * n_ref[...]

    return pl.pallas_call(
        body,
        grid=(_GRID,),
        in_specs=[
            pl.BlockSpec((_BLK, D_H), lambda i: (i, 0)),
            pl.BlockSpec((_BLK, 1), lambda i: (i, 0)),
        ],
        out_specs=pl.BlockSpec((_BLK, D_H), lambda i: (i, 0)),
        out_shape=jax.ShapeDtypeStruct((N, D_H), jnp.float32),
    )(xw, nsrc)


def _tc_mid(parts, nsrc, ndst, b1, w2):
    def body(p_ref, ns_ref, nd_ref, b_ref, w_ref, o_ref):
        agg = jnp.concatenate([p_ref[0], p_ref[1]], axis=-1)
        h = jnp.maximum(agg * nd_ref[...] + b_ref[...], 0.0)
        o_ref[...] = _dot(h * ns_ref[...], w_ref[...])

    return pl.pallas_call(
        body,
        grid=(_GRID,),
        in_specs=[
            pl.BlockSpec((NC, _BLK, D_H // 2), lambda i: (0, i, 0)),
            pl.BlockSpec((_BLK, 1), lambda i: (i, 0)),
            pl.BlockSpec((_BLK, 1), lambda i: (i, 0)),
            pl.BlockSpec((1, D_H), lambda i: (0, 0)),
            pl.BlockSpec((D_H, D_OUT), lambda i: (0, 0)),
        ],
        out_specs=pl.BlockSpec((_BLK, D_OUT), lambda i: (i, 0)),
        out_shape=jax.ShapeDtypeStruct((N, D_OUT), jnp.float32),
    )(parts, nsrc, ndst, b1, w2)


def _tc_final(parts, ndst, b2):
    def body(p_ref, nd_ref, b_ref, o_ref):
        agg = p_ref[0] + p_ref[1]
        o_ref[...] = agg * nd_ref[...] + b_ref[...]

    return pl.pallas_call(
        body,
        grid=(_GRID,),
        in_specs=[
            pl.BlockSpec((NC, _BLK, D_OUT), lambda i: (0, i, 0)),
            pl.BlockSpec((_BLK, 1), lambda i: (i, 0)),
            pl.BlockSpec((1, D_OUT), lambda i: (0, 0)),
        ],
        out_specs=pl.BlockSpec((_BLK, D_OUT), lambda i: (i, 0)),
        out_shape=jax.ShapeDtypeStruct((N, D_OUT), jnp.float32),
    )(parts, ndst, b2)


def kernel(features, edge_index, W1, b1, W2, b2):
    ei = edge_index.reshape(2, NCHUNKS, CHUNK)

    dego, degi = _sc_degrees(ei)
    xw1 = _tc_matmul(features, W1)
    nsrc, ndst = _tc_norms(dego, degi)

    table1 = _tc_scale(xw1, nsrc)
    parts1 = _sc_aggregate_h(table1.reshape(2 * N, D_H // 2), ei)
    table2 = _tc_mid(parts1, nsrc, ndst,
                     b1.reshape(1, D_H), W2)
    parts2 = _sc_aggregate_o(table2, ei)
    return _tc_final(parts2, ndst, b2.reshape(1, D_OUT))


# speedup vs baseline: 14.9598x; 1.0277x over previous
import dataclasses
import functools

import jax
import jax.numpy as jnp
from jax import lax
from jax.experimental import pallas as pl
from jax.experimental.pallas import tpu as pltpu
from jax.experimental.pallas import tpu_sc as plsc

N = 10000
E = 320000
D_IN = 128
D_H = 128
D_OUT = 64

NC = 2
NS = 16
L = 16
NW = NC * NS

CHUNK = 128
NCHUNKS = E // CHUNK
NPAD = 10240
ROWS_PER_TILE = NPAD // NS

_vector_mesh = plsc.VectorSubcoreMesh(core_axis_name="c", subcore_axis_name="s")

_sc_params = pltpu.CompilerParams()
if "needs_layout_passes" in pltpu.CompilerParams.__dataclass_fields__:
    _sc_params = dataclasses.replace(_sc_params, needs_layout_passes=False)
_sc_agg_params = dataclasses.replace(_sc_params, use_tc_tiling_on_sc=False,
                                     internal_scratch_in_bytes=262144)


NPAD_DEG = 10240
DCPT = NCHUNKS // NW
DREM = (NCHUNKS % NW) // 2


@functools.partial(
    pl.kernel,
    out_type=(
        jax.ShapeDtypeStruct((NW, NPAD_DEG), jnp.float32),
        jax.ShapeDtypeStruct((NW, NPAD_DEG), jnp.float32),
    ),
    mesh=_vector_mesh,
    scratch_types=[
        pltpu.VMEM((NPAD_DEG,), jnp.float32),
        pltpu.VMEM((NPAD_DEG,), jnp.float32),
        pltpu.VMEM((DCPT + 2, CHUNK), jnp.int32),
        pltpu.SemaphoreType.DMA,
    ],
    compiler_params=_sc_agg_params,
)
def _sc_degrees(ei_hbm, dego_hbm, degi_hbm, ho, hi, idxb, sem):
    cid = lax.axis_index("c")
    sid = lax.axis_index("s")
    wid = cid * NS + sid

    zeros16 = jnp.zeros((L,), jnp.float32)
    ones16 = jnp.ones((L,), jnp.float32)

    @pl.loop(0, NPAD_DEG // L, unroll=8)
    def _(i):
        ho[pl.ds(i * L, L)] = zeros16
        hi[pl.ds(i * L, L)] = zeros16

    start = DCPT * wid + 2 * jnp.minimum(wid, DREM)
    ncw = DCPT + jnp.where(wid < DREM, 2, 0)

    for e, hist in ((0, ho), (1, hi)):
        pltpu.sync_copy(ei_hbm.at[e, pl.ds(start, DCPT)],
                        idxb.at[pl.ds(0, DCPT)])

        @pl.when(ncw > DCPT)
        def _():
            pltpu.sync_copy(ei_hbm.at[e, pl.ds(start + DCPT, 2)],
                            idxb.at[pl.ds(DCPT, 2)])

        @pl.loop(0, ncw)
        def _(r):
            @pl.loop(0, CHUNK // L, unroll=8)
            def _(i):
                plsc.addupdate_scatter(hist, [idxb[r, pl.ds(i * L, L)]], ones16)

    pltpu.sync_copy(ho, dego_hbm.at[wid])
    pltpu.sync_copy(hi, degi_hbm.at[wid])


def _make_sc_aggregate(D, col_split):
    nsplit = NS if col_split else NW
    cpt = NCHUNKS // nsplit
    rem = NCHUNKS % nsplit
    assert cpt % 2 == 0 and rem % 2 == 0
    nxt = rem // 2

    @functools.partial(
        pl.kernel,
        out_type=jax.ShapeDtypeStruct((NC, NPAD, D), jnp.float32),
        mesh=_vector_mesh,
        scratch_types=[
            pltpu.VMEM((CHUNK, D), jnp.float32),
            pltpu.VMEM((CHUNK, D), jnp.float32),
            pltpu.VMEM((CHUNK, D), jnp.float32),
            pltpu.VMEM((CHUNK, D), jnp.float32),
            pltpu.VMEM((CHUNK, D), jnp.float32),
            pltpu.VMEM((CHUNK, D), jnp.float32),
            pltpu.VMEM((cpt + 2, CHUNK), jnp.int32),
            pltpu.VMEM((cpt + 2, CHUNK), jnp.int32),
            pltpu.VMEM_SHARED((NPAD, D), jnp.float32),
            pltpu.SemaphoreType.DMA,
            pltpu.SemaphoreType.DMA,
            pltpu.SemaphoreType.DMA,
            pltpu.SemaphoreType.DMA,
            pltpu.SemaphoreType.DMA,
            pltpu.SemaphoreType.DMA,
        ],
        compiler_params=_sc_agg_params,
    )
    def _sc_aggregate(table_hbm, ei_hbm, out_hbm,
                      r0, r1, r2, r3, r4, r5, idxs, idxd, acc,
                      s0, s1, s2, s3, s4, s5):
        cid = lax.axis_index("c")
        sid = lax.axis_index("s")
        w = sid if col_split else cid * NS + sid

        NB = 6
        bufs = (r0, r1, r2, r3, r4, r5)
        sems = (s0, s1, s2, s3, s4, s5)

        zeros16 = jnp.zeros((L,), jnp.float32)

        start = cpt * w + 2 * jnp.minimum(w, nxt)
        has_extra = w < nxt

        pltpu.sync_copy(ei_hbm.at[0, pl.ds(start, cpt)],
                        idxs.at[pl.ds(0, cpt)])
        pltpu.sync_copy(ei_hbm.at[1, pl.ds(start, cpt)],
                        idxd.at[pl.ds(0, cpt)])

        @pl.when(has_extra)
        def _():
            pltpu.sync_copy(ei_hbm.at[0, pl.ds(start + cpt, 2)],
                            idxs.at[pl.ds(cpt, 2)])
            pltpu.sync_copy(ei_hbm.at[1, pl.ds(start + cpt, 2)],
                            idxd.at[pl.ds(cpt, 2)])

        if col_split:
            off = jnp.full((L,), 1, jnp.int32) * cid

            @pl.loop(0, cpt + 2)
            def _(r):
                @pl.loop(0, CHUNK // L, unroll=8)
                def _(j):
                    s = pl.ds(j * L, L)
                    idxs[r, s] = idxs[r, s] + idxs[r, s] + off

        @pl.loop(0, CHUNK)
        def _(r):
            @pl.loop(0, D // L)
            def _(j):
                r0[r, pl.ds(j * L, L)] = zeros16

        base = sid * ROWS_PER_TILE
        for z in range(ROWS_PER_TILE // CHUNK):
            pltpu.sync_copy(r0, acc.at[pl.ds(base + z * CHUNK, CHUNK)])
        plsc.subcore_barrier()

        def gather(c, k, sem=None):
            return pltpu.async_copy(table_hbm.at[idxs.at[c]], bufs[k],
                                    sems[k] if sem is None else sem)

        def wait_gather(c, k):
            pltpu.make_async_copy(table_hbm.at[idxs.at[c]], bufs[k],
                                  sems[k]).wait()

        def scatter_add(c, k):
            pltpu.sync_copy(bufs[k], acc.at[idxd.at[c]], add=True)

        for k in range(NB - 1):
            gather(k, k)

        nloop = cpt // NB
        tail = cpt % NB

        @pl.loop(0, nloop)
        def _(p):
            c = NB * p
            for k in range(NB):
                wait_gather(c + k, k)
                scatter_add(c + k, k)

                @pl.when(c + k + NB - 1 < cpt)
                def _():
                    gather(c + k + NB - 1, (k + NB - 1) % NB)

        cbase = nloop * NB
        for t in range(tail):
            c = cbase + t
            wait_gather(c, t)
            scatter_add(c, t)

        @pl.when(has_extra)
        def _():
            da = gather(cpt, 0)
            db = gather(cpt + 1, 1)
            da.wait()
            scatter_add(cpt, 0)
            db.wait()
            scatter_add(cpt + 1, 1)

        plsc.subcore_barrier()
        pltpu.sync_copy(acc.at[pl.ds(base, ROWS_PER_TILE)],
                        out_hbm.at[cid, pl.ds(base, ROWS_PER_TILE)])

    return _sc_aggregate


_sc_aggregate_h = _make_sc_aggregate(D_H // 2, col_split=True)
_sc_aggregate_o = _make_sc_aggregate(D_OUT, col_split=False)


_BLK = 1000
_GRID = N // _BLK


def _dot(a, b):
    return lax.dot_general(a, b, (((1,), (0,)), ((), ())),
                           preferred_element_type=jnp.float32,
                           precision=lax.Precision.HIGHEST)


def _tc_matmul(x, w):
    k = w.shape[1]

    def body(x_ref, w_ref, o_ref):
        o_ref[...] = _dot(x_ref[...], w_ref[...])

    return pl.pallas_call(
        body,
        grid=(_GRID,),
        in_specs=[
            pl.BlockSpec((_BLK, x.shape[1]), lambda i: (i, 0)),
            pl.BlockSpec((x.shape[1], k), lambda i: (0, 0)),
        ],
        out_specs=pl.BlockSpec((_BLK, k), lambda i: (i, 0)),
        out_shape=jax.ShapeDtypeStruct((N, k), jnp.float32),
    )(x, w)


def _tc_norms(dego_raw, degi_raw):
    blk = 1024
    grid = NPAD_DEG // blk

    def body(do_ref, di_ref, no_ref, ni_ref):
        for d_ref, n_ref in ((do_ref, no_ref), (di_ref, ni_ref)):
            deg = jnp.sum(d_ref[...], axis=0, keepdims=True)
            norm = lax.rsqrt(jnp.maximum(deg, 1.0))
            n_ref[...] = jnp.transpose(norm, (1, 0))

    return pl.pallas_call(
        body,
        grid=(grid,),
        in_specs=[
            pl.BlockSpec((NW, blk), lambda i: (0, i)),
            pl.BlockSpec((NW, blk), lambda i: (0, i)),
        ],
        out_specs=[
            pl.BlockSpec((blk, 1), lambda i: (i, 0)),
            pl.BlockSpec((blk, 1), lambda i: (i, 0)),
        ],
        out_shape=[
            jax.ShapeDtypeStruct((NPAD_DEG, 1), jnp.float32),
            jax.ShapeDtypeStruct((NPAD_DEG, 1), jnp.float32),
        ],
    )(dego_raw, degi_raw)


def _tc_scale(xw, nsrc):
    def body(x_ref, n_ref, o_ref):
        o_ref[...] = x_ref[...] * n_ref[...]

    return pl.pallas_call(
        body,
        grid=(_GRID,),
        in_specs=[
            pl.BlockSpec((_BLK, D_H), lambda i: (i, 0)),
            pl.BlockSpec((_BLK, 1), lambda i: (i, 0)),
        ],
        out_specs=pl.BlockSpec((_BLK, D_H), lambda i: (i, 0)),
        out_shape=jax.ShapeDtypeStruct((N, D_H), jnp.float32),
    )(xw, nsrc)


def _tc_mid(parts, nsrc, ndst, b1, w2):
    def body(p_ref, ns_ref, nd_ref, b_ref, w_ref, o_ref):
        agg = jnp.concatenate([p_ref[0], p_ref[1]], axis=-1)
        h = jnp.maximum(agg * nd_ref[...] + b_ref[...], 0.0)
        o_ref[...] = _dot(h * ns_ref[...], w_ref[...])

    return pl.pallas_call(
        body,
        grid=(_GRID,),
        in_specs=[
            pl.BlockSpec((NC, _BLK, D_H // 2), lambda i: (0, i, 0)),
            pl.BlockSpec((_BLK, 1), lambda i: (i, 0)),
            pl.BlockSpec((_BLK, 1), lambda i: (i, 0)),
            pl.BlockSpec((1, D_H), lambda i: (0, 0)),
            pl.BlockSpec((D_H, D_OUT), lambda i: (0, 0)),
        ],
        out_specs=pl.BlockSpec((_BLK, D_OUT), lambda i: (i, 0)),
        out_shape=jax.ShapeDtypeStruct((N, D_OUT), jnp.float32),
    )(parts, nsrc, ndst, b1, w2)


def _tc_final(parts, ndst, b2):
    def body(p_ref, nd_ref, b_ref, o_ref):
        agg = p_ref[0] + p_ref[1]
        o_ref[...] = agg * nd_ref[...] + b_ref[...]

    return pl.pallas_call(
        body,
        grid=(_GRID,),
        in_specs=[
            pl.BlockSpec((NC, _BLK, D_OUT), lambda i: (0, i, 0)),
            pl.BlockSpec((_BLK, 1), lambda i: (i, 0)),
            pl.BlockSpec((1, D_OUT), lambda i: (0, 0)),
        ],
        out_specs=pl.BlockSpec((_BLK, D_OUT), lambda i: (i, 0)),
        out_shape=jax.ShapeDtypeStruct((N, D_OUT), jnp.float32),
    )(parts, ndst, b2)


def kernel(features, edge_index, W1, b1, W2, b2):
    ei = edge_index.reshape(2, NCHUNKS, CHUNK)

    dego, degi = _sc_degrees(ei)
    xw1 = _tc_matmul(features, W1)
    nsrc, ndst = _tc_norms(dego, degi)

    table1 = _tc_scale(xw1, nsrc)
    parts1 = _sc_aggregate_h(table1.reshape(2 * N, D_H // 2), ei)
    table2 = _tc_mid(parts1, nsrc, ndst,
                     b1.reshape(1, D_H), W2)
    parts2 = _sc_aggregate_o(table2, ei)
    return _tc_final(parts2, ndst, b2.reshape(1, D_OUT))


# speedup vs baseline: 15.6125x; 1.0436x over previous
import dataclasses
import functools

import jax
import jax.numpy as jnp
from jax import lax
from jax.experimental import pallas as pl
from jax.experimental.pallas import tpu as pltpu
from jax.experimental.pallas import tpu_sc as plsc

N = 10000
E = 320000
D_IN = 128
D_H = 128
D_OUT = 64

NC = 2
NS = 16
L = 16
NW = NC * NS

CHUNK = 128
NCHUNKS = E // CHUNK
NPAD = 10240
ROWS_PER_TILE = NPAD // NS

_vector_mesh = plsc.VectorSubcoreMesh(core_axis_name="c", subcore_axis_name="s")

_sc_params = pltpu.CompilerParams()
if "needs_layout_passes" in pltpu.CompilerParams.__dataclass_fields__:
    _sc_params = dataclasses.replace(_sc_params, needs_layout_passes=False)
_sc_agg_params = dataclasses.replace(_sc_params, use_tc_tiling_on_sc=False,
                                     internal_scratch_in_bytes=262144)


NPAD_DEG = 10240
DCPT = NCHUNKS // NW
DREM = (NCHUNKS % NW) // 2


@functools.partial(
    pl.kernel,
    out_type=(
        jax.ShapeDtypeStruct((NW, NPAD_DEG), jnp.float32),
        jax.ShapeDtypeStruct((NW, NPAD_DEG), jnp.float32),
    ),
    mesh=_vector_mesh,
    scratch_types=[
        pltpu.VMEM((NPAD_DEG,), jnp.float32),
        pltpu.VMEM((NPAD_DEG,), jnp.float32),
        pltpu.VMEM((DCPT + 2, CHUNK), jnp.int32),
        pltpu.SemaphoreType.DMA,
    ],
    compiler_params=_sc_agg_params,
)
def _sc_degrees(ei_hbm, dego_hbm, degi_hbm, ho, hi, idxb, sem):
    cid = lax.axis_index("c")
    sid = lax.axis_index("s")
    wid = cid * NS + sid

    zeros16 = jnp.zeros((L,), jnp.float32)
    ones16 = jnp.ones((L,), jnp.float32)

    @pl.loop(0, NPAD_DEG // L, unroll=8)
    def _(i):
        ho[pl.ds(i * L, L)] = zeros16
        hi[pl.ds(i * L, L)] = zeros16

    start = DCPT * wid + 2 * jnp.minimum(wid, DREM)
    ncw = DCPT + jnp.where(wid < DREM, 2, 0)

    for e, hist in ((0, ho), (1, hi)):
        pltpu.sync_copy(ei_hbm.at[e, pl.ds(start, DCPT)],
                        idxb.at[pl.ds(0, DCPT)])

        @pl.when(ncw > DCPT)
        def _():
            pltpu.sync_copy(ei_hbm.at[e, pl.ds(start + DCPT, 2)],
                            idxb.at[pl.ds(DCPT, 2)])

        @pl.loop(0, ncw)
        def _(r):
            @pl.loop(0, CHUNK // L, unroll=8)
            def _(i):
                plsc.addupdate_scatter(hist, [idxb[r, pl.ds(i * L, L)]], ones16)

    pltpu.sync_copy(ho, dego_hbm.at[wid])
    pltpu.sync_copy(hi, degi_hbm.at[wid])


def _make_sc_aggregate(D, col_split):
    nsplit = NS if col_split else NW
    cpt = NCHUNKS // nsplit
    rem = NCHUNKS % nsplit
    assert cpt % 2 == 0 and rem % 2 == 0
    nxt = rem // 2

    @functools.partial(
        pl.kernel,
        out_type=jax.ShapeDtypeStruct((NC, NPAD, D), jnp.float32),
        mesh=_vector_mesh,
        scratch_types=[
            pltpu.VMEM((CHUNK, D), jnp.float32),
            pltpu.VMEM((CHUNK, D), jnp.float32),
            pltpu.VMEM((CHUNK, D), jnp.float32),
            pltpu.VMEM((CHUNK, D), jnp.float32),
            pltpu.VMEM((CHUNK, D), jnp.float32),
            pltpu.VMEM((CHUNK, D), jnp.float32),
            pltpu.VMEM((cpt + 2, CHUNK), jnp.int32),
            pltpu.VMEM((cpt + 2, CHUNK), jnp.int32),
            pltpu.VMEM_SHARED((NPAD, D), jnp.float32),
            pltpu.SemaphoreType.DMA,
            pltpu.SemaphoreType.DMA,
            pltpu.SemaphoreType.DMA,
            pltpu.SemaphoreType.DMA,
            pltpu.SemaphoreType.DMA,
            pltpu.SemaphoreType.DMA,
        ],
        compiler_params=_sc_agg_params,
    )
    def _sc_aggregate(table_hbm, ei_hbm, out_hbm,
                      r0, r1, r2, r3, r4, r5, idxs, idxd, acc,
                      s0, s1, s2, s3, s4, s5):
        cid = lax.axis_index("c")
        sid = lax.axis_index("s")
        w = sid if col_split else cid * NS + sid

        NB = 6
        bufs = (r0, r1, r2, r3, r4, r5)
        sems = (s0, s1, s2, s3, s4, s5)

        zeros16 = jnp.zeros((L,), jnp.float32)

        start = cpt * w + 2 * jnp.minimum(w, nxt)
        has_extra = w < nxt

        pltpu.sync_copy(ei_hbm.at[0, pl.ds(start, cpt)],
                        idxs.at[pl.ds(0, cpt)])
        pltpu.sync_copy(ei_hbm.at[1, pl.ds(start, cpt)],
                        idxd.at[pl.ds(0, cpt)])

        @pl.when(has_extra)
        def _():
            pltpu.sync_copy(ei_hbm.at[0, pl.ds(start + cpt, 2)],
                            idxs.at[pl.ds(cpt, 2)])
            pltpu.sync_copy(ei_hbm.at[1, pl.ds(start + cpt, 2)],
                            idxd.at[pl.ds(cpt, 2)])

        if col_split:
            off = jnp.full((L,), 1, jnp.int32) * cid

            @pl.loop(0, cpt + 2)
            def _(r):
                @pl.loop(0, CHUNK // L, unroll=8)
                def _(j):
                    s = pl.ds(j * L, L)
                    idxs[r, s] = idxs[r, s] + idxs[r, s] + off

        @pl.loop(0, CHUNK)
        def _(r):
            @pl.loop(0, D // L)
            def _(j):
                r0[r, pl.ds(j * L, L)] = zeros16

        base = sid * ROWS_PER_TILE
        for z in range(ROWS_PER_TILE // CHUNK):
            pltpu.sync_copy(r0, acc.at[pl.ds(base + z * CHUNK, CHUNK)])
        plsc.subcore_barrier()

        def gather(c, k, sem=None):
            return pltpu.async_copy(table_hbm.at[idxs.at[c]], bufs[k],
                                    sems[k] if sem is None else sem)

        def wait_gather(c, k):
            pltpu.make_async_copy(table_hbm.at[idxs.at[c]], bufs[k],
                                  sems[k]).wait()

        def scatter_add(c, k):
            pltpu.sync_copy(bufs[k], acc.at[idxd.at[c]], add=True)

        for k in range(NB - 1):
            gather(k, k)

        nloop = cpt // NB
        tail = cpt % NB

        @pl.loop(0, nloop)
        def _(p):
            c = NB * p
            for k in range(NB):
                wait_gather(c + k, k)
                scatter_add(c + k, k)

                @pl.when(c + k + NB - 1 < cpt)
                def _():
                    gather(c + k + NB - 1, (k + NB - 1) % NB)

        cbase = nloop * NB
        for t in range(tail):
            c = cbase + t
            wait_gather(c, t)
            scatter_add(c, t)

        @pl.when(has_extra)
        def _():
            da = gather(cpt, 0)
            db = gather(cpt + 1, 1)
            da.wait()
            scatter_add(cpt, 0)
            db.wait()
            scatter_add(cpt + 1, 1)

        plsc.subcore_barrier()
        pltpu.sync_copy(acc.at[pl.ds(base, ROWS_PER_TILE)],
                        out_hbm.at[cid, pl.ds(base, ROWS_PER_TILE)])

    return _sc_aggregate


_sc_aggregate_h = _make_sc_aggregate(D_H // 2, col_split=True)
_sc_aggregate_o = _make_sc_aggregate(D_OUT, col_split=False)


_BLK = 1000
_GRID = N // _BLK


def _dot(a, b):
    return lax.dot_general(a, b, (((1,), (0,)), ((), ())),
                           preferred_element_type=jnp.float32,
                           precision=lax.Precision.HIGHEST)


def _tc_matmul(x, w):
    k = w.shape[1]

    def body(x_ref, w_ref, o_ref):
        o_ref[...] = _dot(x_ref[...], w_ref[...])

    return pl.pallas_call(
        body,
        grid=(_GRID,),
        in_specs=[
            pl.BlockSpec((_BLK, x.shape[1]), lambda i: (i, 0)),
            pl.BlockSpec((x.shape[1], k), lambda i: (0, 0)),
        ],
        out_specs=pl.BlockSpec((_BLK, k), lambda i: (i, 0)),
        out_shape=jax.ShapeDtypeStruct((N, k), jnp.float32),
    )(x, w)


def _tc_norms(dego_raw, degi_raw):
    blk = 1024

    def body(do_ref, di_ref, nsc_ref, nsp_ref, ndp_ref):
        pairs = []
        for d_ref in (do_ref, di_ref):
            deg = jnp.sum(d_ref[...], axis=0, keepdims=True)
            norm = lax.rsqrt(jnp.maximum(deg, 1.0))
            col = jnp.transpose(norm, (1, 0))
            n2 = col.reshape(blk // 2, 2)
            pairs.append(jnp.concatenate(
                [jnp.broadcast_to(n2[:, 0:1], (blk // 2, 64)),
                 jnp.broadcast_to(n2[:, 1:2], (blk // 2, 64))], axis=1))
            if d_ref is do_ref:
                nsc_ref[...] = col
        nsp_ref[...] = pairs[0]
        ndp_ref[...] = pairs[1]

    return pl.pallas_call(
        body,
        grid=(NPAD_DEG // blk,),
        in_specs=[
            pl.BlockSpec((NW, blk), lambda i: (0, i)),
            pl.BlockSpec((NW, blk), lambda i: (0, i)),
        ],
        out_specs=[
            pl.BlockSpec((blk, 1), lambda i: (i, 0)),
            pl.BlockSpec((blk // 2, 128), lambda i: (i, 0)),
            pl.BlockSpec((blk // 2, 128), lambda i: (i, 0)),
        ],
        out_shape=[
            jax.ShapeDtypeStruct((NPAD_DEG, 1), jnp.float32),
            jax.ShapeDtypeStruct((NPAD_DEG // 2, 128), jnp.float32),
            jax.ShapeDtypeStruct((NPAD_DEG // 2, 128), jnp.float32),
        ],
    )(dego_raw, degi_raw)


def _tc_scale(xw, nsrc):
    def body(x_ref, n_ref, o_ref):
        o_ref[...] = x_ref[...] * n_ref[...]

    return pl.pallas_call(
        body,
        grid=(_GRID,),
        in_specs=[
            pl.BlockSpec((_BLK, D_H), lambda i: (i, 0)),
            pl.BlockSpec((_BLK, 1), lambda i: (i, 0)),
        ],
        out_specs=pl.BlockSpec((_BLK, D_H), lambda i: (i, 0)),
        out_shape=jax.ShapeDtypeStruct((N, D_H), jnp.float32),
    )(xw, nsrc)


_PBLK = 512


def _tc_mid(parts1p, nsp, ndp, b1p0, b1p1, w2da, w2db):
    def body(p_ref, nsp_ref, ndp_ref, b0_ref, b1_ref, da_ref, db_ref, o_ref):
        ndpv = ndp_ref[...]
        nspv = nsp_ref[...]
        hp0 = jnp.maximum(p_ref[0] * ndpv + b0_ref[...], 0.0) * nspv
        hp1 = jnp.maximum(p_ref[1] * ndpv + b1_ref[...], 0.0) * nspv
        o_ref[...] = _dot(hp0, da_ref[...]) + _dot(hp1, db_ref[...])

    return pl.pallas_call(
        body,
        grid=(_GRID,),
        in_specs=[
            pl.BlockSpec((NC, _PBLK, 128), lambda i: (0, i, 0)),
            pl.BlockSpec((_PBLK, 128), lambda i: (i, 0)),
            pl.BlockSpec((_PBLK, 128), lambda i: (i, 0)),
            pl.BlockSpec((1, 128), lambda i: (0, 0)),
            pl.BlockSpec((1, 128), lambda i: (0, 0)),
            pl.BlockSpec((128, 128), lambda i: (0, 0)),
            pl.BlockSpec((128, 128), lambda i: (0, 0)),
        ],
        out_specs=pl.BlockSpec((_PBLK, 128), lambda i: (i, 0)),
        out_shape=jax.ShapeDtypeStruct((NPAD // 2, 128), jnp.float32),
    )(parts1p, nsp, ndp, b1p0, b1p1, w2da, w2db)


def _tc_final(parts2p, ndp, b2p):
    def body(p_ref, nd_ref, b_ref, o_ref):
        agg = p_ref[0] + p_ref[1]
        o_ref[...] = agg * nd_ref[...] + b_ref[...]

    return pl.pallas_call(
        body,
        grid=(_GRID,),
        in_specs=[
            pl.BlockSpec((NC, _PBLK, 128), lambda i: (0, i, 0)),
            pl.BlockSpec((_PBLK, 128), lambda i: (i, 0)),
            pl.BlockSpec((1, 128), lambda i: (0, 0)),
        ],
        out_specs=pl.BlockSpec((_PBLK, 128), lambda i: (i, 0)),
        out_shape=jax.ShapeDtypeStruct((NPAD // 2, 128), jnp.float32),
    )(parts2p, ndp, b2p)


def kernel(features, edge_index, W1, b1, W2, b2):
    ei = edge_index.reshape(2, NCHUNKS, CHUNK)

    w2a, w2b = W2[:64], W2[64:]
    z64 = jnp.zeros((64, D_OUT), jnp.float32)
    w2da = jnp.concatenate(
        [jnp.concatenate([w2a, z64], 1), jnp.concatenate([z64, w2a], 1)], 0)
    w2db = jnp.concatenate(
        [jnp.concatenate([w2b, z64], 1), jnp.concatenate([z64, w2b], 1)], 0)
    b1p0 = jnp.concatenate([b1[:64], b1[:64]]).reshape(1, 128)
    b1p1 = jnp.concatenate([b1[64:], b1[64:]]).reshape(1, 128)
    b2p = jnp.concatenate([b2, b2]).reshape(1, 128)

    dego, degi = _sc_degrees(ei)
    xw1 = _tc_matmul(features, W1)
    nsrc, nsp, ndp = _tc_norms(dego, degi)

    table1 = _tc_scale(xw1, nsrc)
    parts1 = _sc_aggregate_h(table1.reshape(2 * N, D_H // 2), ei)
    table2p = _tc_mid(parts1.reshape(NC, NPAD // 2, 128), nsp, ndp,
                      b1p0, b1p1, w2da, w2db)
    parts2 = _sc_aggregate_o(table2p.reshape(NPAD, D_OUT), ei)
    outp = _tc_final(parts2.reshape(NC, NPAD // 2, 128), ndp, b2p)
    return outp.reshape(NPAD, D_OUT)[:N]


# speedup vs baseline: 16.7234x; 1.0712x over previous
import dataclasses
import functools

import jax
import jax.numpy as jnp
from jax import lax
from jax.experimental import pallas as pl
from jax.experimental.pallas import tpu as pltpu
from jax.experimental.pallas import tpu_sc as plsc

N = 10000
E = 320000
D_IN = 128
D_H = 128
D_OUT = 64

NC = 2
NS = 16
L = 16
NW = NC * NS

CHUNK = 128
NCHUNKS = E // CHUNK
NPAD = 10240
ROWS_PER_TILE = NPAD // NS

_vector_mesh = plsc.VectorSubcoreMesh(core_axis_name="c", subcore_axis_name="s")

_sc_params = pltpu.CompilerParams()
if "needs_layout_passes" in pltpu.CompilerParams.__dataclass_fields__:
    _sc_params = dataclasses.replace(_sc_params, needs_layout_passes=False)
_sc_agg_params = dataclasses.replace(_sc_params, use_tc_tiling_on_sc=False,
                                     internal_scratch_in_bytes=262144)


NPAD_DEG = 10240
DCPT = NCHUNKS // NW
DREM = (NCHUNKS % NW) // 2


@functools.partial(
    pl.kernel,
    out_type=(
        jax.ShapeDtypeStruct((NW, NPAD_DEG), jnp.float32),
        jax.ShapeDtypeStruct((NW, NPAD_DEG), jnp.float32),
    ),
    mesh=_vector_mesh,
    scratch_types=[
        pltpu.VMEM((NPAD_DEG,), jnp.float32),
        pltpu.VMEM((NPAD_DEG,), jnp.float32),
        pltpu.VMEM((DCPT + 2, CHUNK), jnp.int32),
        pltpu.SemaphoreType.DMA,
    ],
    compiler_params=_sc_agg_params,
)
def _sc_degrees(ei_hbm, dego_hbm, degi_hbm, ho, hi, idxb, sem):
    cid = lax.axis_index("c")
    sid = lax.axis_index("s")
    wid = cid * NS + sid

    zeros16 = jnp.zeros((L,), jnp.float32)
    ones16 = jnp.ones((L,), jnp.float32)

    @pl.loop(0, NPAD_DEG // L, unroll=8)
    def _(i):
        ho[pl.ds(i * L, L)] = zeros16
        hi[pl.ds(i * L, L)] = zeros16

    start = DCPT * wid + 2 * jnp.minimum(wid, DREM)
    ncw = DCPT + jnp.where(wid < DREM, 2, 0)

    for e, hist in ((0, ho), (1, hi)):
        pltpu.sync_copy(ei_hbm.at[e, pl.ds(start, DCPT)],
                        idxb.at[pl.ds(0, DCPT)])

        @pl.when(ncw > DCPT)
        def _():
            pltpu.sync_copy(ei_hbm.at[e, pl.ds(start + DCPT, 2)],
                            idxb.at[pl.ds(DCPT, 2)])

        @pl.loop(0, ncw)
        def _(r):
            @pl.loop(0, CHUNK // L, unroll=8)
            def _(i):
                plsc.addupdate_scatter(hist, [idxb[r, pl.ds(i * L, L)]], ones16)

    pltpu.sync_copy(ho, dego_hbm.at[wid])
    pltpu.sync_copy(hi, degi_hbm.at[wid])


def _make_sc_aggregate(D, col_split):
    nsplit = NS if col_split else NW
    cpt = NCHUNKS // nsplit
    rem = NCHUNKS % nsplit
    assert cpt % 2 == 0 and rem % 2 == 0
    nxt = rem // 2

    @functools.partial(
        pl.kernel,
        out_type=jax.ShapeDtypeStruct((NC, NPAD, D), jnp.float32),
        mesh=_vector_mesh,
        scratch_types=[
            pltpu.VMEM((CHUNK, D), jnp.float32),
            pltpu.VMEM((CHUNK, D), jnp.float32),
            pltpu.VMEM((CHUNK, D), jnp.float32),
            pltpu.VMEM((CHUNK, D), jnp.float32),
            pltpu.VMEM((CHUNK, D), jnp.float32),
            pltpu.VMEM((CHUNK, D), jnp.float32),
            pltpu.VMEM((cpt + 2, CHUNK), jnp.int32),
            pltpu.VMEM((cpt + 2, CHUNK), jnp.int32),
            pltpu.VMEM_SHARED((NPAD, D), jnp.float32),
            pltpu.SemaphoreType.DMA,
            pltpu.SemaphoreType.DMA,
            pltpu.SemaphoreType.DMA,
            pltpu.SemaphoreType.DMA,
            pltpu.SemaphoreType.DMA,
            pltpu.SemaphoreType.DMA,
        ],
        compiler_params=_sc_agg_params,
    )
    def _sc_aggregate(table_hbm, ei_hbm, out_hbm,
                      r0, r1, r2, r3, r4, r5, idxs, idxd, acc,
                      s0, s1, s2, s3, s4, s5):
        cid = lax.axis_index("c")
        sid = lax.axis_index("s")
        w = sid if col_split else cid * NS + sid

        NB = 6
        bufs = (r0, r1, r2, r3, r4, r5)
        sems = (s0, s1, s2, s3, s4, s5)

        zeros16 = jnp.zeros((L,), jnp.float32)

        start = cpt * w + 2 * jnp.minimum(w, nxt)
        has_extra = w < nxt

        pltpu.sync_copy(ei_hbm.at[0, pl.ds(start, cpt)],
                        idxs.at[pl.ds(0, cpt)])
        pltpu.sync_copy(ei_hbm.at[1, pl.ds(start, cpt)],
                        idxd.at[pl.ds(0, cpt)])

        @pl.when(has_extra)
        def _():
            pltpu.sync_copy(ei_hbm.at[0, pl.ds(start + cpt, 2)],
                            idxs.at[pl.ds(cpt, 2)])
            pltpu.sync_copy(ei_hbm.at[1, pl.ds(start + cpt, 2)],
                            idxd.at[pl.ds(cpt, 2)])

        if col_split:
            off = jnp.full((L,), 1, jnp.int32) * cid

            @pl.loop(0, cpt + 2)
            def _(r):
                @pl.loop(0, CHUNK // L, unroll=8)
                def _(j):
                    s = pl.ds(j * L, L)
                    idxs[r, s] = idxs[r, s] + idxs[r, s] + off

        @pl.loop(0, CHUNK)
        def _(r):
            @pl.loop(0, D // L)
            def _(j):
                r0[r, pl.ds(j * L, L)] = zeros16

        base = sid * ROWS_PER_TILE
        for z in range(ROWS_PER_TILE // CHUNK):
            pltpu.sync_copy(r0, acc.at[pl.ds(base + z * CHUNK, CHUNK)])
        plsc.subcore_barrier()

        def gather(c, k, sem=None):
            return pltpu.async_copy(table_hbm.at[idxs.at[c]], bufs[k],
                                    sems[k] if sem is None else sem)

        def wait_gather(c, k):
            pltpu.make_async_copy(table_hbm.at[idxs.at[c]], bufs[k],
                                  sems[k]).wait()

        def scatter_add(c, k):
            pltpu.sync_copy(bufs[k], acc.at[idxd.at[c]], add=True)

        for k in range(NB - 1):
            gather(k, k)

        nloop = cpt // NB
        tail = cpt % NB

        @pl.loop(0, nloop)
        def _(p):
            c = NB * p
            for k in range(NB):
                wait_gather(c + k, k)
                scatter_add(c + k, k)

                @pl.when(c + k + NB - 1 < cpt)
                def _():
                    gather(c + k + NB - 1, (k + NB - 1) % NB)

        cbase = nloop * NB
        for t in range(tail):
            c = cbase + t
            wait_gather(c, t)
            scatter_add(c, t)

        @pl.when(has_extra)
        def _():
            da = gather(cpt, 0)
            db = gather(cpt + 1, 1)
            da.wait()
            scatter_add(cpt, 0)
            db.wait()
            scatter_add(cpt + 1, 1)

        plsc.subcore_barrier()
        pltpu.sync_copy(acc.at[pl.ds(base, ROWS_PER_TILE)],
                        out_hbm.at[cid, pl.ds(base, ROWS_PER_TILE)])

    return _sc_aggregate


_sc_aggregate_h = _make_sc_aggregate(D_H // 2, col_split=True)
_sc_aggregate_o = _make_sc_aggregate(D_OUT, col_split=False)


_BLK = 1000
_GRID = N // _BLK


def _dot(a, b):
    return lax.dot_general(a, b, (((1,), (0,)), ((), ())),
                           preferred_element_type=jnp.float32,
                           precision=lax.Precision.HIGHEST)


def _tc_matmul(x, w):
    k = w.shape[1]

    def body(x_ref, w_ref, o_ref):
        o_ref[...] = _dot(x_ref[...], w_ref[...])

    return pl.pallas_call(
        body,
        grid=(_GRID,),
        in_specs=[
            pl.BlockSpec((_BLK, x.shape[1]), lambda i: (i, 0)),
            pl.BlockSpec((x.shape[1], k), lambda i: (0, 0)),
        ],
        out_specs=pl.BlockSpec((_BLK, k), lambda i: (i, 0)),
        out_shape=jax.ShapeDtypeStruct((N, k), jnp.float32),
    )(x, w)


def _tc_norms(dego_raw, degi_raw):
    blk = 1024

    def body(do_ref, di_ref, no_ref, ni_ref):
        for d_ref, n_ref in ((do_ref, no_ref), (di_ref, ni_ref)):
            deg = jnp.sum(d_ref[...], axis=0, keepdims=True)
            norm = lax.rsqrt(jnp.maximum(deg, 1.0))
            n_ref[...] = jnp.transpose(norm, (1, 0))

    return pl.pallas_call(
        body,
        grid=(NPAD_DEG // blk,),
        in_specs=[
            pl.BlockSpec((NW, blk), lambda i: (0, i)),
            pl.BlockSpec((NW, blk), lambda i: (0, i)),
        ],
        out_specs=[
            pl.BlockSpec((blk, 1), lambda i: (i, 0)),
            pl.BlockSpec((blk, 1), lambda i: (i, 0)),
        ],
        out_shape=[
            jax.ShapeDtypeStruct((NPAD_DEG, 1), jnp.float32),
            jax.ShapeDtypeStruct((NPAD_DEG, 1), jnp.float32),
        ],
    )(dego_raw, degi_raw)


def _pair_expand(col):
    n2 = col.reshape(NPAD_DEG // 2, 2)
    return jnp.concatenate([jnp.repeat(n2[:, :1], 64, axis=1),
                            jnp.repeat(n2[:, 1:], 64, axis=1)], axis=1)


def _tc_scale(xw, nsrc):
    def body(x_ref, n_ref, o_ref):
        o_ref[...] = x_ref[...] * n_ref[...]

    return pl.pallas_call(
        body,
        grid=(_GRID,),
        in_specs=[
            pl.BlockSpec((_BLK, D_H), lambda i: (i, 0)),
            pl.BlockSpec((_BLK, 1), lambda i: (i, 0)),
        ],
        out_specs=pl.BlockSpec((_BLK, D_H), lambda i: (i, 0)),
        out_shape=jax.ShapeDtypeStruct((N, D_H), jnp.float32),
    )(xw, nsrc)


_PBLK = 512


def _tc_mid(parts1p, nsp, ndp, b1p0, b1p1, w2da, w2db):
    def body(p_ref, nsp_ref, ndp_ref, b0_ref, b1_ref, da_ref, db_ref, o_ref):
        ndpv = ndp_ref[...]
        nspv = nsp_ref[...]
        hp0 = jnp.maximum(p_ref[0] * ndpv + b0_ref[...], 0.0) * nspv
        hp1 = jnp.maximum(p_ref[1] * ndpv + b1_ref[...], 0.0) * nspv
        o_ref[...] = _dot(hp0, da_ref[...]) + _dot(hp1, db_ref[...])

    return pl.pallas_call(
        body,
        grid=(_GRID,),
        in_specs=[
            pl.BlockSpec((NC, _PBLK, 128), lambda i: (0, i, 0)),
            pl.BlockSpec((_PBLK, 128), lambda i: (i, 0)),
            pl.BlockSpec((_PBLK, 128), lambda i: (i, 0)),
            pl.BlockSpec((1, 128), lambda i: (0, 0)),
            pl.BlockSpec((1, 128), lambda i: (0, 0)),
            pl.BlockSpec((128, 128), lambda i: (0, 0)),
            pl.BlockSpec((128, 128), lambda i: (0, 0)),
        ],
        out_specs=pl.BlockSpec((_PBLK, 128), lambda i: (i, 0)),
        out_shape=jax.ShapeDtypeStruct((NPAD // 2, 128), jnp.float32),
    )(parts1p, nsp, ndp, b1p0, b1p1, w2da, w2db)


def _tc_final(parts2p, ndp, b2p):
    def body(p_ref, nd_ref, b_ref, o_ref):
        agg = p_ref[0] + p_ref[1]
        o_ref[...] = agg * nd_ref[...] + b_ref[...]

    return pl.pallas_call(
        body,
        grid=(_GRID,),
        in_specs=[
            pl.BlockSpec((NC, _PBLK, 128), lambda i: (0, i, 0)),
            pl.BlockSpec((_PBLK, 128), lambda i: (i, 0)),
            pl.BlockSpec((1, 128), lambda i: (0, 0)),
        ],
        out_specs=pl.BlockSpec((_PBLK, 128), lambda i: (i, 0)),
        out_shape=jax.ShapeDtypeStruct((NPAD // 2, 128), jnp.float32),
    )(parts2p, ndp, b2p)


def kernel(features, edge_index, W1, b1, W2, b2):
    ei = edge_index.reshape(2, NCHUNKS, CHUNK)

    w2a, w2b = W2[:64], W2[64:]
    z64 = jnp.zeros((64, D_OUT), jnp.float32)
    w2da = jnp.concatenate(
        [jnp.concatenate([w2a, z64], 1), jnp.concatenate([z64, w2a], 1)], 0)
    w2db = jnp.concatenate(
        [jnp.concatenate([w2b, z64], 1), jnp.concatenate([z64, w2b], 1)], 0)
    b1p0 = jnp.concatenate([b1[:64], b1[:64]]).reshape(1, 128)
    b1p1 = jnp.concatenate([b1[64:], b1[64:]]).reshape(1, 128)
    b2p = jnp.concatenate([b2, b2]).reshape(1, 128)

    dego, degi = _sc_degrees(ei)
    xw1 = _tc_matmul(features, W1)
    nsrc, ndst = _tc_norms(dego, degi)
    nsp, ndp = _pair_expand(nsrc), _pair_expand(ndst)

    table1 = _tc_scale(xw1, nsrc)
    parts1 = _sc_aggregate_h(table1.reshape(2 * N, D_H // 2), ei)
    table2p = _tc_mid(parts1.reshape(NC, NPAD // 2, 128), nsp, ndp,
                      b1p0, b1p1, w2da, w2db)
    parts2 = _sc_aggregate_o(table2p.reshape(NPAD, D_OUT), ei)
    outp = _tc_final(parts2.reshape(NC, NPAD // 2, 128), ndp, b2p)
    return outp.reshape(NPAD, D_OUT)[:N]


# speedup vs baseline: 17.2995x; 1.0344x over previous
import dataclasses
import functools

import jax
import jax.numpy as jnp
from jax import lax
from jax.experimental import pallas as pl
from jax.experimental.pallas import tpu as pltpu
from jax.experimental.pallas import tpu_sc as plsc

N = 10000
E = 320000
D_IN = 128
D_H = 128
D_OUT = 64

NC = 2
NS = 16
L = 16
NW = NC * NS

CHUNK = 128
NCHUNKS = E // CHUNK
NPAD = 10240
ROWS_PER_TILE = NPAD // NS

_vector_mesh = plsc.VectorSubcoreMesh(core_axis_name="c", subcore_axis_name="s")

_sc_params = pltpu.CompilerParams()
if "needs_layout_passes" in pltpu.CompilerParams.__dataclass_fields__:
    _sc_params = dataclasses.replace(_sc_params, needs_layout_passes=False)
_sc_agg_params = dataclasses.replace(_sc_params, use_tc_tiling_on_sc=False,
                                     internal_scratch_in_bytes=262144)


NPAD_DEG = 10240
DCPT = NCHUNKS // NW
DREM = (NCHUNKS % NW) // 2


@functools.partial(
    pl.kernel,
    out_type=(
        jax.ShapeDtypeStruct((NW, NPAD_DEG), jnp.float32),
        jax.ShapeDtypeStruct((NW, NPAD_DEG), jnp.float32),
    ),
    mesh=_vector_mesh,
    scratch_types=[
        pltpu.VMEM((NPAD_DEG,), jnp.float32),
        pltpu.VMEM((NPAD_DEG,), jnp.float32),
        pltpu.VMEM((DCPT + 2, CHUNK), jnp.int32),
        pltpu.SemaphoreType.DMA,
    ],
    compiler_params=_sc_agg_params,
)
def _sc_degrees(ei_hbm, dego_hbm, degi_hbm, ho, hi, idxb, sem):
    cid = lax.axis_index("c")
    sid = lax.axis_index("s")
    wid = cid * NS + sid

    zeros16 = jnp.zeros((L,), jnp.float32)
    ones16 = jnp.ones((L,), jnp.float32)

    @pl.loop(0, NPAD_DEG // L, unroll=8)
    def _(i):
        ho[pl.ds(i * L, L)] = zeros16
        hi[pl.ds(i * L, L)] = zeros16

    start = DCPT * wid + 2 * jnp.minimum(wid, DREM)
    ncw = DCPT + jnp.where(wid < DREM, 2, 0)

    for e, hist in ((0, ho), (1, hi)):
        pltpu.sync_copy(ei_hbm.at[e, pl.ds(start, DCPT)],
                        idxb.at[pl.ds(0, DCPT)])

        @pl.when(ncw > DCPT)
        def _():
            pltpu.sync_copy(ei_hbm.at[e, pl.ds(start + DCPT, 2)],
                            idxb.at[pl.ds(DCPT, 2)])

        @pl.loop(0, ncw)
        def _(r):
            @pl.loop(0, CHUNK // L, unroll=8)
            def _(i):
                plsc.addupdate_scatter(hist, [idxb[r, pl.ds(i * L, L)]], ones16)

    pltpu.sync_copy(ho, dego_hbm.at[wid])
    pltpu.sync_copy(hi, degi_hbm.at[wid])


def _make_sc_aggregate(D, col_split):
    nsplit = NS if col_split else NW
    cpt = NCHUNKS // nsplit
    rem = NCHUNKS % nsplit
    assert cpt % 2 == 0 and rem % 2 == 0
    nxt = rem // 2

    @functools.partial(
        pl.kernel,
        out_type=jax.ShapeDtypeStruct((NC, NPAD, D), jnp.float32),
        mesh=_vector_mesh,
        scratch_types=[
            pltpu.VMEM((CHUNK, D), jnp.float32),
            pltpu.VMEM((CHUNK, D), jnp.float32),
            pltpu.VMEM((CHUNK, D), jnp.float32),
            pltpu.VMEM((CHUNK, D), jnp.float32),
            pltpu.VMEM((CHUNK, D), jnp.float32),
            pltpu.VMEM((CHUNK, D), jnp.float32),
            pltpu.VMEM((cpt + 2, CHUNK), jnp.int32),
            pltpu.VMEM((cpt + 2, CHUNK), jnp.int32),
            pltpu.VMEM_SHARED((NPAD, D), jnp.float32),
            pltpu.SemaphoreType.DMA,
            pltpu.SemaphoreType.DMA,
            pltpu.SemaphoreType.DMA,
            pltpu.SemaphoreType.DMA,
            pltpu.SemaphoreType.DMA,
            pltpu.SemaphoreType.DMA,
        ],
        compiler_params=_sc_agg_params,
    )
    def _sc_aggregate(table_hbm, ei_hbm, out_hbm,
                      r0, r1, r2, r3, r4, r5, idxs, idxd, acc,
                      s0, s1, s2, s3, s4, s5):
        cid = lax.axis_index("c")
        sid = lax.axis_index("s")
        w = sid if col_split else cid * NS + sid

        NB = 6
        bufs = (r0, r1, r2, r3, r4, r5)
        sems = (s0, s1, s2, s3, s4, s5)

        zeros16 = jnp.zeros((L,), jnp.float32)

        start = cpt * w + 2 * jnp.minimum(w, nxt)
        has_extra = w < nxt

        pltpu.sync_copy(ei_hbm.at[0, pl.ds(start, cpt)],
                        idxs.at[pl.ds(0, cpt)])
        pltpu.sync_copy(ei_hbm.at[1, pl.ds(start, cpt)],
                        idxd.at[pl.ds(0, cpt)])

        @pl.when(has_extra)
        def _():
            pltpu.sync_copy(ei_hbm.at[0, pl.ds(start + cpt, 2)],
                            idxs.at[pl.ds(cpt, 2)])
            pltpu.sync_copy(ei_hbm.at[1, pl.ds(start + cpt, 2)],
                            idxd.at[pl.ds(cpt, 2)])

        if col_split:
            off = jnp.full((L,), 1, jnp.int32) * cid

            @pl.loop(0, cpt + 2)
            def _(r):
                @pl.loop(0, CHUNK // L, unroll=8)
                def _(j):
                    s = pl.ds(j * L, L)
                    idxs[r, s] = idxs[r, s] + idxs[r, s] + off

        @pl.loop(0, CHUNK)
        def _(r):
            @pl.loop(0, D // L)
            def _(j):
                r0[r, pl.ds(j * L, L)] = zeros16

        base = sid * ROWS_PER_TILE
        for z in range(ROWS_PER_TILE // CHUNK):
            pltpu.sync_copy(r0, acc.at[pl.ds(base + z * CHUNK, CHUNK)])
        plsc.subcore_barrier()

        def gather(c, k, sem=None):
            return pltpu.async_copy(table_hbm.at[idxs.at[c]], bufs[k],
                                    sems[k] if sem is None else sem)

        def wait_gather(c, k):
            pltpu.make_async_copy(table_hbm.at[idxs.at[c]], bufs[k],
                                  sems[k]).wait()

        def scatter_add(c, k):
            pltpu.sync_copy(bufs[k], acc.at[idxd.at[c]], add=True)

        for k in range(NB - 1):
            gather(k, k)

        nloop = cpt // NB
        tail = cpt % NB

        @pl.loop(0, nloop)
        def _(p):
            c = NB * p
            for k in range(NB):
                wait_gather(c + k, k)
                scatter_add(c + k, k)

                @pl.when(c + k + NB - 1 < cpt)
                def _():
                    gather(c + k + NB - 1, (k + NB - 1) % NB)

        cbase = nloop * NB
        for t in range(tail):
            c = cbase + t
            wait_gather(c, t)
            scatter_add(c, t)

        @pl.when(has_extra)
        def _():
            da = gather(cpt, 0)
            db = gather(cpt + 1, 1)
            da.wait()
            scatter_add(cpt, 0)
            db.wait()
            scatter_add(cpt + 1, 1)

        plsc.subcore_barrier()
        pltpu.sync_copy(acc.at[pl.ds(base, ROWS_PER_TILE)],
                        out_hbm.at[cid, pl.ds(base, ROWS_PER_TILE)])

    return _sc_aggregate


_sc_aggregate_h = _make_sc_aggregate(D_H // 2, col_split=True)
_sc_aggregate_o = _make_sc_aggregate(D_OUT, col_split=False)


_BLK = 1000
_GRID = N // _BLK


def _dot(a, b):
    return lax.dot_general(a, b, (((1,), (0,)), ((), ())),
                           preferred_element_type=jnp.float32,
                           precision=lax.Precision.HIGHEST)


def _tc_matmul(x, w):
    k = w.shape[1]

    def body(x_ref, w_ref, o_ref):
        o_ref[...] = _dot(x_ref[...], w_ref[...])

    return pl.pallas_call(
        body,
        grid=(_GRID,),
        in_specs=[
            pl.BlockSpec((_BLK, x.shape[1]), lambda i: (i, 0)),
            pl.BlockSpec((x.shape[1], k), lambda i: (0, 0)),
        ],
        out_specs=pl.BlockSpec((_BLK, k), lambda i: (i, 0)),
        out_shape=jax.ShapeDtypeStruct((N, k), jnp.float32),
    )(x, w)


def _tc_norms(dego_raw, degi_raw):
    blk = 2048

    def body(do_ref, di_ref, no_ref, ni_ref):
        for d_ref, n_ref in ((do_ref, no_ref), (di_ref, ni_ref)):
            deg = jnp.sum(d_ref[...], axis=0, keepdims=True)
            norm = lax.rsqrt(jnp.maximum(deg, 1.0))
            n_ref[...] = jnp.transpose(norm, (1, 0))

    return pl.pallas_call(
        body,
        grid=(NPAD_DEG // blk,),
        in_specs=[
            pl.BlockSpec((NW, blk), lambda i: (0, i)),
            pl.BlockSpec((NW, blk), lambda i: (0, i)),
        ],
        out_specs=[
            pl.BlockSpec((blk, 1), lambda i: (i, 0)),
            pl.BlockSpec((blk, 1), lambda i: (i, 0)),
        ],
        out_shape=[
            jax.ShapeDtypeStruct((NPAD_DEG, 1), jnp.float32),
            jax.ShapeDtypeStruct((NPAD_DEG, 1), jnp.float32),
        ],
    )(dego_raw, degi_raw)


def _pair_expand(col):
    n2 = col.reshape(NPAD_DEG // 2, 2)
    return jnp.concatenate([jnp.repeat(n2[:, :1], 64, axis=1),
                            jnp.repeat(n2[:, 1:], 64, axis=1)], axis=1)


def _tc_scale(xw, nsrc):
    blk = 2000

    def body(x_ref, n_ref, o_ref):
        o_ref[...] = x_ref[...] * n_ref[...]

    return pl.pallas_call(
        body,
        grid=(N // blk,),
        in_specs=[
            pl.BlockSpec((blk, D_H), lambda i: (i, 0)),
            pl.BlockSpec((blk, 1), lambda i: (i, 0)),
        ],
        out_specs=pl.BlockSpec((blk, D_H), lambda i: (i, 0)),
        out_shape=jax.ShapeDtypeStruct((N, D_H), jnp.float32),
    )(xw, nsrc)


_PBLK = 1024


def _tc_mid(parts1p, nsp, ndp, b1p0, b1p1, w2da, w2db):
    def body(p_ref, nsp_ref, ndp_ref, b0_ref, b1_ref, da_ref, db_ref, o_ref):
        ndpv = ndp_ref[...]
        nspv = nsp_ref[...]
        hp0 = jnp.maximum(p_ref[0] * ndpv + b0_ref[...], 0.0) * nspv
        hp1 = jnp.maximum(p_ref[1] * ndpv + b1_ref[...], 0.0) * nspv
        o_ref[...] = _dot(hp0, da_ref[...]) + _dot(hp1, db_ref[...])

    return pl.pallas_call(
        body,
        grid=(NPAD // 2 // _PBLK,),
        in_specs=[
            pl.BlockSpec((NC, _PBLK, 128), lambda i: (0, i, 0)),
            pl.BlockSpec((_PBLK, 128), lambda i: (i, 0)),
            pl.BlockSpec((_PBLK, 128), lambda i: (i, 0)),
            pl.BlockSpec((1, 128), lambda i: (0, 0)),
            pl.BlockSpec((1, 128), lambda i: (0, 0)),
            pl.BlockSpec((128, 128), lambda i: (0, 0)),
            pl.BlockSpec((128, 128), lambda i: (0, 0)),
        ],
        out_specs=pl.BlockSpec((_PBLK, 128), lambda i: (i, 0)),
        out_shape=jax.ShapeDtypeStruct((NPAD // 2, 128), jnp.float32),
    )(parts1p, nsp, ndp, b1p0, b1p1, w2da, w2db)


def _tc_final(parts2p, ndp, b2p):
    def body(p_ref, nd_ref, b_ref, o_ref):
        agg = p_ref[0] + p_ref[1]
        o_ref[...] = agg * nd_ref[...] + b_ref[...]

    return pl.pallas_call(
        body,
        grid=(NPAD // 2 // _PBLK,),
        in_specs=[
            pl.BlockSpec((NC, _PBLK, 128), lambda i: (0, i, 0)),
            pl.BlockSpec((_PBLK, 128), lambda i: (i, 0)),
            pl.BlockSpec((1, 128), lambda i: (0, 0)),
        ],
        out_specs=pl.BlockSpec((_PBLK, 128), lambda i: (i, 0)),
        out_shape=jax.ShapeDtypeStruct((NPAD // 2, 128), jnp.float32),
    )(parts2p, ndp, b2p)


def kernel(features, edge_index, W1, b1, W2, b2):
    ei = edge_index.reshape(2, NCHUNKS, CHUNK)

    w2a, w2b = W2[:64], W2[64:]
    z64 = jnp.zeros((64, D_OUT), jnp.float32)
    w2da = jnp.concatenate(
        [jnp.concatenate([w2a, z64], 1), jnp.concatenate([z64, w2a], 1)], 0)
    w2db = jnp.concatenate(
        [jnp.concatenate([w2b, z64], 1), jnp.concatenate([z64, w2b], 1)], 0)
    b1p0 = jnp.concatenate([b1[:64], b1[:64]]).reshape(1, 128)
    b1p1 = jnp.concatenate([b1[64:], b1[64:]]).reshape(1, 128)
    b2p = jnp.concatenate([b2, b2]).reshape(1, 128)

    dego, degi = _sc_degrees(ei)
    xw1 = _tc_matmul(features, W1)
    nsrc, ndst = _tc_norms(dego, degi)
    nsp, ndp = _pair_expand(nsrc), _pair_expand(ndst)

    table1 = _tc_scale(xw1, nsrc)
    parts1 = _sc_aggregate_h(table1.reshape(2 * N, D_H // 2), ei)
    table2p = _tc_mid(parts1.reshape(NC, NPAD // 2, 128), nsp, ndp,
                      b1p0, b1p1, w2da, w2db)
    parts2 = _sc_aggregate_o(table2p.reshape(NPAD, D_OUT), ei)
    outp = _tc_final(parts2.reshape(NC, NPAD // 2, 128), ndp, b2p)
    return outp[:N // 2].reshape(N, D_OUT)


# speedup vs baseline: 17.3093x; 1.0006x over previous
import dataclasses
import functools

import jax
import jax.numpy as jnp
from jax import lax
from jax.experimental import pallas as pl
from jax.experimental.pallas import tpu as pltpu
from jax.experimental.pallas import tpu_sc as plsc

N = 10000
E = 320000
D_IN = 128
D_H = 128
D_OUT = 64

NC = 2
NS = 16
L = 16
NW = NC * NS

CHUNK = 128
NCHUNKS = E // CHUNK
NPAD = 10240
ROWS_PER_TILE = NPAD // NS

_vector_mesh = plsc.VectorSubcoreMesh(core_axis_name="c", subcore_axis_name="s")

_sc_params = pltpu.CompilerParams()
if "needs_layout_passes" in pltpu.CompilerParams.__dataclass_fields__:
    _sc_params = dataclasses.replace(_sc_params, needs_layout_passes=False)
_sc_agg_params = dataclasses.replace(_sc_params, use_tc_tiling_on_sc=False,
                                     internal_scratch_in_bytes=262144)


NPAD_DEG = 10240
DCPT = NCHUNKS // NW
DREM = (NCHUNKS % NW) // 2


@functools.partial(
    pl.kernel,
    out_type=(
        jax.ShapeDtypeStruct((NW, NPAD_DEG), jnp.float32),
        jax.ShapeDtypeStruct((NW, NPAD_DEG), jnp.float32),
    ),
    mesh=_vector_mesh,
    scratch_types=[
        pltpu.VMEM((NPAD_DEG,), jnp.float32),
        pltpu.VMEM((NPAD_DEG,), jnp.float32),
        pltpu.VMEM((DCPT + 2, CHUNK), jnp.int32),
        pltpu.SemaphoreType.DMA,
    ],
    compiler_params=_sc_agg_params,
)
def _sc_degrees(ei_hbm, dego_hbm, degi_hbm, ho, hi, idxb, sem):
    cid = lax.axis_index("c")
    sid = lax.axis_index("s")
    wid = cid * NS + sid

    zeros16 = jnp.zeros((L,), jnp.float32)
    ones16 = jnp.ones((L,), jnp.float32)

    @pl.loop(0, NPAD_DEG // L, unroll=8)
    def _(i):
        ho[pl.ds(i * L, L)] = zeros16
        hi[pl.ds(i * L, L)] = zeros16

    start = DCPT * wid + 2 * jnp.minimum(wid, DREM)
    ncw = DCPT + jnp.where(wid < DREM, 2, 0)

    for e, hist in ((0, ho), (1, hi)):
        pltpu.sync_copy(ei_hbm.at[e, pl.ds(start, DCPT)],
                        idxb.at[pl.ds(0, DCPT)])

        @pl.when(ncw > DCPT)
        def _():
            pltpu.sync_copy(ei_hbm.at[e, pl.ds(start + DCPT, 2)],
                            idxb.at[pl.ds(DCPT, 2)])

        @pl.loop(0, ncw)
        def _(r):
            @pl.loop(0, CHUNK // L, unroll=8)
            def _(i):
                plsc.addupdate_scatter(hist, [idxb[r, pl.ds(i * L, L)]], ones16)

    pltpu.sync_copy(ho, dego_hbm.at[wid])
    pltpu.sync_copy(hi, degi_hbm.at[wid])


def _make_sc_aggregate(D, col_split):
    nsplit = NS if col_split else NW
    cpt = NCHUNKS // nsplit
    rem = NCHUNKS % nsplit
    assert cpt % 2 == 0 and rem % 2 == 0
    nxt = rem // 2

    @functools.partial(
        pl.kernel,
        out_type=jax.ShapeDtypeStruct((NC, NPAD, D), jnp.float32),
        mesh=_vector_mesh,
        scratch_types=[
            pltpu.VMEM((CHUNK, D), jnp.float32),
            pltpu.VMEM((CHUNK, D), jnp.float32),
            pltpu.VMEM((CHUNK, D), jnp.float32),
            pltpu.VMEM((CHUNK, D), jnp.float32),
            pltpu.VMEM((CHUNK, D), jnp.float32),
            pltpu.VMEM((CHUNK, D), jnp.float32),
            pltpu.VMEM((CHUNK, D), jnp.float32),
            pltpu.VMEM((CHUNK, D), jnp.float32),
            pltpu.VMEM((cpt + 2, CHUNK), jnp.int32),
            pltpu.VMEM((cpt + 2, CHUNK), jnp.int32),
            pltpu.VMEM_SHARED((NPAD, D), jnp.float32),
            pltpu.SemaphoreType.DMA,
            pltpu.SemaphoreType.DMA,
            pltpu.SemaphoreType.DMA,
            pltpu.SemaphoreType.DMA,
            pltpu.SemaphoreType.DMA,
            pltpu.SemaphoreType.DMA,
            pltpu.SemaphoreType.DMA,
            pltpu.SemaphoreType.DMA,
        ],
        compiler_params=_sc_agg_params,
    )
    def _sc_aggregate(table_hbm, ei_hbm, out_hbm,
                      r0, r1, r2, r3, r4, r5, r6, r7, idxs, idxd, acc,
                      s0, s1, s2, s3, s4, s5, s6, s7):
        cid = lax.axis_index("c")
        sid = lax.axis_index("s")
        w = sid if col_split else cid * NS + sid

        NB = 6 if cpt % 8 else 8
        bufs = (r0, r1, r2, r3, r4, r5, r6, r7)[:NB]
        sems = (s0, s1, s2, s3, s4, s5, s6, s7)[:NB]

        zeros16 = jnp.zeros((L,), jnp.float32)

        start = cpt * w + 2 * jnp.minimum(w, nxt)
        has_extra = w < nxt

        pltpu.sync_copy(ei_hbm.at[0, pl.ds(start, cpt)],
                        idxs.at[pl.ds(0, cpt)])
        pltpu.sync_copy(ei_hbm.at[1, pl.ds(start, cpt)],
                        idxd.at[pl.ds(0, cpt)])

        @pl.when(has_extra)
        def _():
            pltpu.sync_copy(ei_hbm.at[0, pl.ds(start + cpt, 2)],
                            idxs.at[pl.ds(cpt, 2)])
            pltpu.sync_copy(ei_hbm.at[1, pl.ds(start + cpt, 2)],
                            idxd.at[pl.ds(cpt, 2)])

        if col_split:
            off = jnp.full((L,), 1, jnp.int32) * cid

            @pl.loop(0, cpt + 2)
            def _(r):
                @pl.loop(0, CHUNK // L, unroll=8)
                def _(j):
                    s = pl.ds(j * L, L)
                    idxs[r, s] = idxs[r, s] + idxs[r, s] + off

        @pl.loop(0, CHUNK)
        def _(r):
            @pl.loop(0, D // L)
            def _(j):
                r0[r, pl.ds(j * L, L)] = zeros16

        base = sid * ROWS_PER_TILE
        for z in range(ROWS_PER_TILE // CHUNK):
            pltpu.sync_copy(r0, acc.at[pl.ds(base + z * CHUNK, CHUNK)])
        plsc.subcore_barrier()

        def gather(c, k, sem=None):
            return pltpu.async_copy(table_hbm.at[idxs.at[c]], bufs[k],
                                    sems[k] if sem is None else sem)

        def wait_gather(c, k):
            pltpu.make_async_copy(table_hbm.at[idxs.at[c]], bufs[k],
                                  sems[k]).wait()

        def scatter_add(c, k):
            pltpu.sync_copy(bufs[k], acc.at[idxd.at[c]], add=True)

        for k in range(NB - 1):
            gather(k, k)

        nloop = cpt // NB
        tail = cpt % NB

        @pl.loop(0, nloop)
        def _(p):
            c = NB * p
            for k in range(NB):
                wait_gather(c + k, k)
                scatter_add(c + k, k)

                @pl.when(c + k + NB - 1 < cpt)
                def _():
                    gather(c + k + NB - 1, (k + NB - 1) % NB)

        cbase = nloop * NB
        for t in range(tail):
            c = cbase + t
            wait_gather(c, t)
            scatter_add(c, t)

        @pl.when(has_extra)
        def _():
            da = gather(cpt, 0)
            db = gather(cpt + 1, 1)
            da.wait()
            scatter_add(cpt, 0)
            db.wait()
            scatter_add(cpt + 1, 1)

        plsc.subcore_barrier()
        pltpu.sync_copy(acc.at[pl.ds(base, ROWS_PER_TILE)],
                        out_hbm.at[cid, pl.ds(base, ROWS_PER_TILE)])

    return _sc_aggregate


_sc_aggregate_h = _make_sc_aggregate(D_H // 2, col_split=True)
_sc_aggregate_o = _make_sc_aggregate(D_OUT, col_split=False)


_BLK = 1000
_GRID = N // _BLK


def _dot(a, b):
    return lax.dot_general(a, b, (((1,), (0,)), ((), ())),
                           preferred_element_type=jnp.float32,
                           precision=lax.Precision.HIGHEST)


def _tc_matmul(x, w):
    k = w.shape[1]

    def body(x_ref, w_ref, o_ref):
        o_ref[...] = _dot(x_ref[...], w_ref[...])

    return pl.pallas_call(
        body,
        grid=(_GRID,),
        in_specs=[
            pl.BlockSpec((_BLK, x.shape[1]), lambda i: (i, 0)),
            pl.BlockSpec((x.shape[1], k), lambda i: (0, 0)),
        ],
        out_specs=pl.BlockSpec((_BLK, k), lambda i: (i, 0)),
        out_shape=jax.ShapeDtypeStruct((N, k), jnp.float32),
    )(x, w)


def _tc_norms(dego_raw, degi_raw):
    blk = 2048

    def body(do_ref, di_ref, no_ref, ni_ref):
        for d_ref, n_ref in ((do_ref, no_ref), (di_ref, ni_ref)):
            deg = jnp.sum(d_ref[...], axis=0, keepdims=True)
            norm = lax.rsqrt(jnp.maximum(deg, 1.0))
            n_ref[...] = jnp.transpose(norm, (1, 0))

    return pl.pallas_call(
        body,
        grid=(NPAD_DEG // blk,),
        in_specs=[
            pl.BlockSpec((NW, blk), lambda i: (0, i)),
            pl.BlockSpec((NW, blk), lambda i: (0, i)),
        ],
        out_specs=[
            pl.BlockSpec((blk, 1), lambda i: (i, 0)),
            pl.BlockSpec((blk, 1), lambda i: (i, 0)),
        ],
        out_shape=[
            jax.ShapeDtypeStruct((NPAD_DEG, 1), jnp.float32),
            jax.ShapeDtypeStruct((NPAD_DEG, 1), jnp.float32),
        ],
    )(dego_raw, degi_raw)


def _pair_expand(col):
    n2 = col.reshape(NPAD_DEG // 2, 2)
    return jnp.concatenate([jnp.repeat(n2[:, :1], 64, axis=1),
                            jnp.repeat(n2[:, 1:], 64, axis=1)], axis=1)


def _tc_scale(xw, nsrc):
    blk = 2000

    def body(x_ref, n_ref, o_ref):
        o_ref[...] = x_ref[...] * n_ref[...]

    return pl.pallas_call(
        body,
        grid=(N // blk,),
        in_specs=[
            pl.BlockSpec((blk, D_H), lambda i: (i, 0)),
            pl.BlockSpec((blk, 1), lambda i: (i, 0)),
        ],
        out_specs=pl.BlockSpec((blk, D_H), lambda i: (i, 0)),
        out_shape=jax.ShapeDtypeStruct((N, D_H), jnp.float32),
    )(xw, nsrc)


_PBLK = 1024


def _tc_mid(parts1p, nsp, ndp, b1p0, b1p1, w2da, w2db):
    def body(p_ref, nsp_ref, ndp_ref, b0_ref, b1_ref, da_ref, db_ref, o_ref):
        ndpv = ndp_ref[...]
        nspv = nsp_ref[...]
        hp0 = jnp.maximum(p_ref[0] * ndpv + b0_ref[...], 0.0) * nspv
        hp1 = jnp.maximum(p_ref[1] * ndpv + b1_ref[...], 0.0) * nspv
        o_ref[...] = _dot(hp0, da_ref[...]) + _dot(hp1, db_ref[...])

    return pl.pallas_call(
        body,
        grid=(NPAD // 2 // _PBLK,),
        in_specs=[
            pl.BlockSpec((NC, _PBLK, 128), lambda i: (0, i, 0)),
            pl.BlockSpec((_PBLK, 128), lambda i: (i, 0)),
            pl.BlockSpec((_PBLK, 128), lambda i: (i, 0)),
            pl.BlockSpec((1, 128), lambda i: (0, 0)),
            pl.BlockSpec((1, 128), lambda i: (0, 0)),
            pl.BlockSpec((128, 128), lambda i: (0, 0)),
            pl.BlockSpec((128, 128), lambda i: (0, 0)),
        ],
        out_specs=pl.BlockSpec((_PBLK, 128), lambda i: (i, 0)),
        out_shape=jax.ShapeDtypeStruct((NPAD // 2, 128), jnp.float32),
    )(parts1p, nsp, ndp, b1p0, b1p1, w2da, w2db)


def _tc_final(parts2p, ndp, b2p):
    def body(p_ref, nd_ref, b_ref, o_ref):
        agg = p_ref[0] + p_ref[1]
        o_ref[...] = agg * nd_ref[...] + b_ref[...]

    return pl.pallas_call(
        body,
        grid=(NPAD // 2 // _PBLK,),
        in_specs=[
            pl.BlockSpec((NC, _PBLK, 128), lambda i: (0, i, 0)),
            pl.BlockSpec((_PBLK, 128), lambda i: (i, 0)),
            pl.BlockSpec((1, 128), lambda i: (0, 0)),
        ],
        out_specs=pl.BlockSpec((_PBLK, 128), lambda i: (i, 0)),
        out_shape=jax.ShapeDtypeStruct((NPAD // 2, 128), jnp.float32),
    )(parts2p, ndp, b2p)


def kernel(features, edge_index, W1, b1, W2, b2):
    ei = edge_index.reshape(2, NCHUNKS, CHUNK)

    w2a, w2b = W2[:64], W2[64:]
    z64 = jnp.zeros((64, D_OUT), jnp.float32)
    w2da = jnp.concatenate(
        [jnp.concatenate([w2a, z64], 1), jnp.concatenate([z64, w2a], 1)], 0)
    w2db = jnp.concatenate(
        [jnp.concatenate([w2b, z64], 1), jnp.concatenate([z64, w2b], 1)], 0)
    b1p0 = jnp.concatenate([b1[:64], b1[:64]]).reshape(1, 128)
    b1p1 = jnp.concatenate([b1[64:], b1[64:]]).reshape(1, 128)
    b2p = jnp.concatenate([b2, b2]).reshape(1, 128)

    dego, degi = _sc_degrees(ei)
    xw1 = _tc_matmul(features, W1)
    nsrc, ndst = _tc_norms(dego, degi)
    nsp, ndp = _pair_expand(nsrc), _pair_expand(ndst)

    table1 = _tc_scale(xw1, nsrc)
    parts1 = _sc_aggregate_h(table1.reshape(2 * N, D_H // 2), ei)
    table2p = _tc_mid(parts1.reshape(NC, NPAD // 2, 128), nsp, ndp,
                      b1p0, b1p1, w2da, w2db)
    parts2 = _sc_aggregate_o(table2p.reshape(NPAD, D_OUT), ei)
    outp = _tc_final(parts2.reshape(NC, NPAD // 2, 128), ndp, b2p)
    return outp[:N // 2].reshape(N, D_OUT)


# speedup vs baseline: 17.3283x; 1.0011x over previous
import dataclasses
import functools

import jax
import jax.numpy as jnp
from jax import lax
from jax.experimental import pallas as pl
from jax.experimental.pallas import tpu as pltpu
from jax.experimental.pallas import tpu_sc as plsc

N = 10000
E = 320000
D_IN = 128
D_H = 128
D_OUT = 64

NC = 2
NS = 16
L = 16
NW = NC * NS

CHUNK = 128
NCHUNKS = E // CHUNK
NPAD = 10240
ROWS_PER_TILE = NPAD // NS

_vector_mesh = plsc.VectorSubcoreMesh(core_axis_name="c", subcore_axis_name="s")

_sc_params = pltpu.CompilerParams()
if "needs_layout_passes" in pltpu.CompilerParams.__dataclass_fields__:
    _sc_params = dataclasses.replace(_sc_params, needs_layout_passes=False)
_sc_agg_params = dataclasses.replace(_sc_params, use_tc_tiling_on_sc=False,
                                     internal_scratch_in_bytes=262144)


NPAD_DEG = 10240
DCPT = NCHUNKS // NW
DREM = (NCHUNKS % NW) // 2


@functools.partial(
    pl.kernel,
    out_type=(
        jax.ShapeDtypeStruct((NW, NPAD_DEG), jnp.float32),
        jax.ShapeDtypeStruct((NW, NPAD_DEG), jnp.float32),
    ),
    mesh=_vector_mesh,
    scratch_types=[
        pltpu.VMEM((NPAD_DEG,), jnp.float32),
        pltpu.VMEM((NPAD_DEG,), jnp.float32),
        pltpu.VMEM((DCPT + 2, CHUNK), jnp.int32),
        pltpu.SemaphoreType.DMA,
    ],
    compiler_params=_sc_agg_params,
)
def _sc_degrees(ei_hbm, dego_hbm, degi_hbm, ho, hi, idxb, sem):
    cid = lax.axis_index("c")
    sid = lax.axis_index("s")
    wid = cid * NS + sid

    zeros16 = jnp.zeros((L,), jnp.float32)
    ones16 = jnp.ones((L,), jnp.float32)

    @pl.loop(0, NPAD_DEG // L, unroll=8)
    def _(i):
        ho[pl.ds(i * L, L)] = zeros16
        hi[pl.ds(i * L, L)] = zeros16

    start = DCPT * wid + 2 * jnp.minimum(wid, DREM)
    ncw = DCPT + jnp.where(wid < DREM, 2, 0)

    for e, hist in ((0, ho), (1, hi)):
        pltpu.sync_copy(ei_hbm.at[e, pl.ds(start, DCPT)],
                        idxb.at[pl.ds(0, DCPT)])

        @pl.when(ncw > DCPT)
        def _():
            pltpu.sync_copy(ei_hbm.at[e, pl.ds(start + DCPT, 2)],
                            idxb.at[pl.ds(DCPT, 2)])

        @pl.loop(0, ncw)
        def _(r):
            @pl.loop(0, CHUNK // L, unroll=8)
            def _(i):
                plsc.addupdate_scatter(hist, [idxb[r, pl.ds(i * L, L)]], ones16)

    pltpu.sync_copy(ho, dego_hbm.at[wid])
    pltpu.sync_copy(hi, degi_hbm.at[wid])


def _make_sc_aggregate(D, col_split):
    nsplit = NS if col_split else NW
    cpt = NCHUNKS // nsplit
    rem = NCHUNKS % nsplit
    assert cpt % 2 == 0 and rem % 2 == 0
    nxt = rem // 2

    @functools.partial(
        pl.kernel,
        out_type=jax.ShapeDtypeStruct((NC, NPAD, D), jnp.float32),
        mesh=_vector_mesh,
        scratch_types=[
            pltpu.VMEM((CHUNK, D), jnp.float32),
            pltpu.VMEM((CHUNK, D), jnp.float32),
            pltpu.VMEM((CHUNK, D), jnp.float32),
            pltpu.VMEM((CHUNK, D), jnp.float32),
            pltpu.VMEM((CHUNK, D), jnp.float32),
            pltpu.VMEM((CHUNK, D), jnp.float32),
            pltpu.VMEM((cpt + 2, CHUNK), jnp.int32),
            pltpu.VMEM((cpt + 2, CHUNK), jnp.int32),
            pltpu.VMEM_SHARED((NPAD, D), jnp.float32),
            pltpu.SemaphoreType.DMA,
            pltpu.SemaphoreType.DMA,
            pltpu.SemaphoreType.DMA,
            pltpu.SemaphoreType.DMA,
            pltpu.SemaphoreType.DMA,
            pltpu.SemaphoreType.DMA,
        ],
        compiler_params=_sc_agg_params,
    )
    def _sc_aggregate(table_hbm, ei_hbm, out_hbm,
                      r0, r1, r2, r3, r4, r5, idxs, idxd, acc,
                      s0, s1, s2, s3, s4, s5):
        cid = lax.axis_index("c")
        sid = lax.axis_index("s")
        w = sid if col_split else cid * NS + sid

        NB = 6
        bufs = (r0, r1, r2, r3, r4, r5)
        sems = (s0, s1, s2, s3, s4, s5)

        zeros16 = jnp.zeros((L,), jnp.float32)

        start = cpt * w + 2 * jnp.minimum(w, nxt)
        has_extra = w < nxt

        pltpu.sync_copy(ei_hbm.at[0, pl.ds(start, cpt)],
                        idxs.at[pl.ds(0, cpt)])
        pltpu.sync_copy(ei_hbm.at[1, pl.ds(start, cpt)],
                        idxd.at[pl.ds(0, cpt)])

        @pl.when(has_extra)
        def _():
            pltpu.sync_copy(ei_hbm.at[0, pl.ds(start + cpt, 2)],
                            idxs.at[pl.ds(cpt, 2)])
            pltpu.sync_copy(ei_hbm.at[1, pl.ds(start + cpt, 2)],
                            idxd.at[pl.ds(cpt, 2)])

        if col_split:
            off = jnp.full((L,), 1, jnp.int32) * cid

            @pl.loop(0, cpt + 2)
            def _(r):
                @pl.loop(0, CHUNK // L, unroll=8)
                def _(j):
                    s = pl.ds(j * L, L)
                    idxs[r, s] = idxs[r, s] + idxs[r, s] + off

        @pl.loop(0, CHUNK)
        def _(r):
            @pl.loop(0, D // L)
            def _(j):
                r0[r, pl.ds(j * L, L)] = zeros16

        base = sid * ROWS_PER_TILE
        for z in range(ROWS_PER_TILE // CHUNK):
            pltpu.sync_copy(r0, acc.at[pl.ds(base + z * CHUNK, CHUNK)])
        plsc.subcore_barrier()

        def gather(c, k, sem=None):
            return pltpu.async_copy(table_hbm.at[idxs.at[c]], bufs[k],
                                    sems[k] if sem is None else sem)

        def wait_gather(c, k):
            pltpu.make_async_copy(table_hbm.at[idxs.at[c]], bufs[k],
                                  sems[k]).wait()

        def scatter_add(c, k):
            pltpu.sync_copy(bufs[k], acc.at[idxd.at[c]], add=True)

        for k in range(NB - 1):
            gather(k, k)

        nloop = cpt // NB
        tail = cpt % NB

        @pl.loop(0, nloop)
        def _(p):
            c = NB * p
            for k in range(NB):
                wait_gather(c + k, k)
                scatter_add(c + k, k)

                @pl.when(c + k + NB - 1 < cpt)
                def _():
                    gather(c + k + NB - 1, (k + NB - 1) % NB)

        cbase = nloop * NB
        for t in range(tail):
            c = cbase + t
            wait_gather(c, t)
            scatter_add(c, t)

        @pl.when(has_extra)
        def _():
            da = gather(cpt, 0)
            db = gather(cpt + 1, 1)
            da.wait()
            scatter_add(cpt, 0)
            db.wait()
            scatter_add(cpt + 1, 1)

        plsc.subcore_barrier()
        pltpu.sync_copy(acc.at[pl.ds(base, ROWS_PER_TILE)],
                        out_hbm.at[cid, pl.ds(base, ROWS_PER_TILE)])

    return _sc_aggregate


_sc_aggregate_h = _make_sc_aggregate(D_H // 2, col_split=True)
_sc_aggregate_o = _make_sc_aggregate(D_OUT, col_split=False)


_BLK = 1000
_GRID = N // _BLK


def _dot(a, b):
    return lax.dot_general(a, b, (((1,), (0,)), ((), ())),
                           preferred_element_type=jnp.float32,
                           precision=lax.Precision.HIGHEST)


def _tc_matmul(x, w):
    k = w.shape[1]

    def body(x_ref, w_ref, o_ref):
        o_ref[...] = _dot(x_ref[...], w_ref[...])

    return pl.pallas_call(
        body,
        grid=(_GRID,),
        in_specs=[
            pl.BlockSpec((_BLK, x.shape[1]), lambda i: (i, 0)),
            pl.BlockSpec((x.shape[1], k), lambda i: (0, 0)),
        ],
        out_specs=pl.BlockSpec((_BLK, k), lambda i: (i, 0)),
        out_shape=jax.ShapeDtypeStruct((N, k), jnp.float32),
    )(x, w)


def _tc_norms(dego_raw, degi_raw):
    blk = 2048

    def body(do_ref, di_ref, no_ref, ni_ref):
        for d_ref, n_ref in ((do_ref, no_ref), (di_ref, ni_ref)):
            deg = jnp.sum(d_ref[...], axis=0, keepdims=True)
            norm = lax.rsqrt(jnp.maximum(deg, 1.0))
            n_ref[...] = jnp.transpose(norm, (1, 0))

    return pl.pallas_call(
        body,
        grid=(NPAD_DEG // blk,),
        in_specs=[
            pl.BlockSpec((NW, blk), lambda i: (0, i)),
            pl.BlockSpec((NW, blk), lambda i: (0, i)),
        ],
        out_specs=[
            pl.BlockSpec((blk, 1), lambda i: (i, 0)),
            pl.BlockSpec((blk, 1), lambda i: (i, 0)),
        ],
        out_shape=[
            jax.ShapeDtypeStruct((NPAD_DEG, 1), jnp.float32),
            jax.ShapeDtypeStruct((NPAD_DEG, 1), jnp.float32),
        ],
    )(dego_raw, degi_raw)


def _pair_expand(col):
    n2 = col.reshape(NPAD_DEG // 2, 2)
    return jnp.concatenate([jnp.repeat(n2[:, :1], 64, axis=1),
                            jnp.repeat(n2[:, 1:], 64, axis=1)], axis=1)


def _tc_scale(xw, nsrc):
    blk = 2000

    def body(x_ref, n_ref, o_ref):
        o_ref[...] = x_ref[...] * n_ref[...]

    return pl.pallas_call(
        body,
        grid=(N // blk,),
        in_specs=[
            pl.BlockSpec((blk, D_H), lambda i: (i, 0)),
            pl.BlockSpec((blk, 1), lambda i: (i, 0)),
        ],
        out_specs=pl.BlockSpec((blk, D_H), lambda i: (i, 0)),
        out_shape=jax.ShapeDtypeStruct((N, D_H), jnp.float32),
    )(xw, nsrc)


_PBLK = 1024


def _tc_mid(parts1p, nsp, ndp, b1p0, b1p1, w2da, w2db):
    def body(p_ref, nsp_ref, ndp_ref, b0_ref, b1_ref, da_ref, db_ref, o_ref):
        ndpv = ndp_ref[...]
        nspv = nsp_ref[...]
        hp0 = jnp.maximum(p_ref[0] * ndpv + b0_ref[...], 0.0) * nspv
        hp1 = jnp.maximum(p_ref[1] * ndpv + b1_ref[...], 0.0) * nspv
        o_ref[...] = _dot(hp0, da_ref[...]) + _dot(hp1, db_ref[...])

    return pl.pallas_call(
        body,
        grid=(NPAD // 2 // _PBLK,),
        in_specs=[
            pl.BlockSpec((NC, _PBLK, 128), lambda i: (0, i, 0)),
            pl.BlockSpec((_PBLK, 128), lambda i: (i, 0)),
            pl.BlockSpec((_PBLK, 128), lambda i: (i, 0)),
            pl.BlockSpec((1, 128), lambda i: (0, 0)),
            pl.BlockSpec((1, 128), lambda i: (0, 0)),
            pl.BlockSpec((128, 128), lambda i: (0, 0)),
            pl.BlockSpec((128, 128), lambda i: (0, 0)),
        ],
        out_specs=pl.BlockSpec((_PBLK, 128), lambda i: (i, 0)),
        out_shape=jax.ShapeDtypeStruct((NPAD // 2, 128), jnp.float32),
    )(parts1p, nsp, ndp, b1p0, b1p1, w2da, w2db)


def _tc_final(parts2p, ndp, b2p):
    def body(p_ref, nd_ref, b_ref, o_ref):
        agg = p_ref[0] + p_ref[1]
        o_ref[...] = agg * nd_ref[...] + b_ref[...]

    return pl.pallas_call(
        body,
        grid=(NPAD // 2 // _PBLK,),
        in_specs=[
            pl.BlockSpec((NC, _PBLK, 128), lambda i: (0, i, 0)),
            pl.BlockSpec((_PBLK, 128), lambda i: (i, 0)),
            pl.BlockSpec((1, 128), lambda i: (0, 0)),
        ],
        out_specs=pl.BlockSpec((_PBLK, 128), lambda i: (i, 0)),
        out_shape=jax.ShapeDtypeStruct((NPAD // 2, 128), jnp.float32),
    )(parts2p, ndp, b2p)


def kernel(features, edge_index, W1, b1, W2, b2):
    ei = edge_index.reshape(2, NCHUNKS, CHUNK)

    w2a, w2b = W2[:64], W2[64:]
    z64 = jnp.zeros((64, D_OUT), jnp.float32)
    w2da = jnp.concatenate(
        [jnp.concatenate([w2a, z64], 1), jnp.concatenate([z64, w2a], 1)], 0)
    w2db = jnp.concatenate(
        [jnp.concatenate([w2b, z64], 1), jnp.concatenate([z64, w2b], 1)], 0)
    b1p0 = jnp.concatenate([b1[:64], b1[:64]]).reshape(1, 128)
    b1p1 = jnp.concatenate([b1[64:], b1[64:]]).reshape(1, 128)
    b2p = jnp.concatenate([b2, b2]).reshape(1, 128)

    dego, degi = _sc_degrees(ei)
    xw1 = _tc_matmul(features, W1)
    nsrc, ndst = _tc_norms(dego, degi)
    nsp, ndp = _pair_expand(nsrc), _pair_expand(ndst)

    table1 = _tc_scale(xw1, nsrc)
    parts1 = _sc_aggregate_h(table1.reshape(2 * N, D_H // 2), ei)
    table2p = _tc_mid(parts1.reshape(NC, NPAD // 2, 128), nsp, ndp,
                      b1p0, b1p1, w2da, w2db)
    parts2 = _sc_aggregate_o(table2p.reshape(NPAD, D_OUT), ei)
    outp = _tc_final(parts2.reshape(NC, NPAD // 2, 128), ndp, b2p)
    return outp[:N // 2].reshape(N, D_OUT)
